# baseline (speedup 1.0000x reference)
import functools

import jax
import jax.numpy as jnp
from jax import lax
from jax.experimental import pallas as pl
from jax.experimental.pallas import tpu as pltpu
from jax.experimental.pallas import tpu_sc as plsc

E = 64
TOPK = 2
D = 768
F = 256
SEQ = 2048
TRIM = 32
NS = SEQ - TRIM
NA = NS * TOPK
BLK = 64
NB = 128
P = NB * BLK
TB = 8
NTB = NS // TB
NAP = 4096


def _grouped_mlp_body(be_ref, xg_ref, w1_ref, w2_ref, out_ref):
    h = jax.nn.gelu(
        jnp.dot(xg_ref[...], w1_ref[0], preferred_element_type=jnp.float32)
    )
    out_ref[...] = jnp.dot(h.astype(jnp.bfloat16), w2_ref[0],
                           preferred_element_type=jnp.float32
                           ).astype(jnp.bfloat16)


def _sparse_out_body(rows_ref, w_ref, eid_ref, y_ref, buf_ref):
    rows = rows_ref[...].astype(jnp.float32)
    w = w_ref[...]
    acc = rows[:, 0, :] * w[:, 0:1] + rows[:, 1, :] * w[:, 1:2]
    y_ref[...] = acc
    buf_ref[...] = jnp.zeros((TB, E, D), jnp.float32)
    for t in range(TB):
        e0 = eid_ref[t, 0]
        e1 = eid_ref[t, 1]
        dup = e0 == e1
        both = rows[t, 0, :] + rows[t, 1, :]
        v0 = jnp.where(dup, both, rows[t, 0, :])
        v1 = jnp.where(dup, both, rows[t, 1, :])
        buf_ref[t, e0, :] = v0
        buf_ref[t, e1, :] = v1


EG = 8


def _dense_body(xd_ref, w1_ref, w2_ref, sc_ref, _buf_in, y_ref, buf_ref):
    eg = pl.program_id(0)
    xd = xd_ref[...].astype(jnp.bfloat16)
    r = jax.lax.broadcasted_iota(jnp.int32, (E, EG), 0)
    c = jax.lax.broadcasted_iota(jnp.int32, (E, EG), 1)
    onehot = (r == EG * eg + c).astype(jnp.float32)
    w8 = jnp.dot(sc_ref[...], onehot, preferred_element_type=jnp.float32)

    outs = []
    acc = jnp.zeros((TRIM, D), jnp.float32)
    for j in range(EG):
        h = jax.nn.gelu(
            jnp.dot(xd, w1_ref[j], preferred_element_type=jnp.float32))
        out = jnp.dot(h.astype(jnp.bfloat16), w2_ref[j],
                      preferred_element_type=jnp.float32)
        outs.append(out)
        acc += out * w8[:, j:j + 1]
    buf_ref[...] = jnp.stack(outs, axis=1)

    @pl.when(eg == 0)
    def _():
        y_ref[...] = jnp.zeros((TRIM, D), jnp.float32)

    y_ref[...] += acc


def _exclusive_cumsum(v):
    return jnp.concatenate([jnp.zeros((1,), v.dtype), jnp.cumsum(v)[:-1]])


_NV = NA // 16
_LANE = None


def _sc_meta_body(eid_hbm, bexp_hbm, inv_hbm, gtok_hbm,
                  ev, hist, base, bstart, counters, tmp, inv_v, gtok_v,
                  bexp_v, occ_v, last_v):
    wid = lax.axis_index("s") * 2 + lax.axis_index("c")

    @pl.when(wid == 0)
    def _():
        iota = lax.iota(jnp.int32, 16)
        zeros16 = jnp.zeros((16,), jnp.int32)
        pltpu.sync_copy(eid_hbm, ev)

        def _z(i, _):
            gtok_v[pl.ds(i * 16, 16)] = zeros16
            return 0
        lax.fori_loop(0, P // 16, _z, 0)
        for vi in range(4):
            hist[pl.ds(vi * 16, 16)] = zeros16
            counters[pl.ds(vi * 16, 16)] = zeros16
        tmp[pl.ds(0, 16)] = zeros16
        tmp[pl.ds(16, 16)] = zeros16
        tmp[pl.ds(32, 16)] = zeros16
        for vi in range((NAP - NA) // 16):
            inv_v[pl.ds(NA + vi * 16, 16)] = zeros16

        def _hist_step(i, _):
            v = ev[pl.ds(i * 16, 16)]

            def _dup(m, carry):
                occ, cntl = carry
                sp = plsc.load_gather(ev, [jnp.zeros((16,), jnp.int32)
                                           + (i * 16 + m)])
                eq = v == sp
                occ = occ + jnp.where(eq & (iota > m), 1, 0)
                cntl = cntl + jnp.where(eq & (iota < m), 1, 0)
                return occ, cntl
            occ, cntl = lax.fori_loop(0, 16, _dup, (zeros16, zeros16))
            is_last = cntl == 0
            prior = plsc.load_gather(hist, [v])
            plsc.store_scatter(hist, [v], prior + occ + 1, mask=is_last)
            occ_v[pl.ds(i * 16, 16)] = occ
            last_v[pl.ds(i * 16, 16)] = jnp.where(is_last, 1, 0)
            return 0
        lax.fori_loop(0, _NV, _hist_step, 0)

        def _incl_scan(vec):
            cur = vec
            for sh in (1, 2, 4, 8):
                tmp[pl.ds(16, 16)] = cur
                cur = cur + tmp[pl.ds(16 - sh, 16)]
            tmp[pl.ds(32, 16)] = cur
            return cur

        carry_p = jnp.zeros((16,), jnp.int32)
        carry_b = jnp.zeros((16,), jnp.int32)
        full15 = jnp.full((16,), 15, jnp.int32)
        for vi in range(4):
            h = hist[pl.ds(vi * 16, 16)]
            nb = (h + 63) >> 6
            pad = nb * BLK
            incl_p = _incl_scan(pad)
            base[pl.ds(vi * 16, 16)] = incl_p - pad + carry_p
            carry_p = carry_p + plsc.load_gather(tmp, [full15 + 32])
            incl_b = _incl_scan(nb)
            bstart[pl.ds(vi * 16, 16)] = incl_b - nb + carry_b
            carry_b = carry_b + plsc.load_gather(tmp, [full15 + 32])

        for bi in range(NB // 16):
            bvec = bi * 16 + iota

            def _acc(e, a):
                bs_e = plsc.load_gather(bstart, [jnp.zeros((16,), jnp.int32)
                                                 + e])
                return a + jnp.where(bvec >= bs_e, 1, 0)
            bexp_v[pl.ds(bi * 16, 16)] = lax.fori_loop(
                0, E, _acc, jnp.full((16,), -1, jnp.int32))

        def _slot_step(i, _):
            v = ev[pl.ds(i * 16, 16)]
            occ = occ_v[pl.ds(i * 16, 16)]
            is_last = last_v[pl.ds(i * 16, 16)] == 1
            prior = plsc.load_gather(counters, [v])
            rank = prior + occ
            plsc.store_scatter(counters, [v], rank + 1, mask=is_last)
            slot = plsc.load_gather(base, [v]) + rank
            jg = i * 16 + iota
            inv_v[pl.ds(i * 16, 16)] = slot
            plsc.store_scatter(gtok_v, [slot], jg >> 1)
            return 0
        lax.fori_loop(0, _NV, _slot_step, 0)

        pltpu.sync_copy(bexp_v, bexp_hbm)
        pltpu.sync_copy(inv_v, inv_hbm)
        pltpu.sync_copy(gtok_v, gtok_hbm)


def _sc_row_gather(table_i32, idx, chunk):
    n_rows, d = idx.shape[0], table_i32.shape[1]
    nw = 32
    per_w = n_rows // nw
    nch = per_w // chunk
    mesh = plsc.VectorSubcoreMesh(core_axis_name="c", subcore_axis_name="s")

    def body(table_hbm, idx_hbm, out_hbm, idx_v, rows_v, sem):
        wid = lax.axis_index("s") * 2 + lax.axis_index("c")
        base = wid * per_w
        for c in range(nch):
            b = base + c * chunk
            pltpu.sync_copy(idx_hbm.at[pl.ds(b, chunk)], idx_v)
            pltpu.async_copy(table_hbm.at[idx_v], rows_v, sem).wait()
            pltpu.sync_copy(rows_v, out_hbm.at[pl.ds(b, chunk)])

    k = functools.partial(
        pl.kernel, mesh=mesh,
        out_type=jax.ShapeDtypeStruct((n_rows, d), jnp.int32),
        scratch_types=[
            pltpu.VMEM((chunk,), jnp.int32),
            pltpu.VMEM((chunk, d), jnp.int32),
            pltpu.SemaphoreType.DMA,
        ],
    )(body)
    return k(table_i32, idx)


def _bf16_to_i32(x):
    return lax.bitcast_convert_type(
        x.reshape(*x.shape[:-1], x.shape[-1] // 2, 2), jnp.int32)


def _i32_to_bf16(x):
    y = lax.bitcast_convert_type(x, jnp.bfloat16)
    return y.reshape(*x.shape[:-1], x.shape[-1] * 2)


def _sc_metadata(eid_flat):
    mesh = plsc.VectorSubcoreMesh(core_axis_name="c", subcore_axis_name="s")
    k = functools.partial(
        pl.kernel, mesh=mesh,
        compiler_params=pltpu.CompilerParams(needs_layout_passes=False),
        out_type=[
            jax.ShapeDtypeStruct((NB,), jnp.int32),
            jax.ShapeDtypeStruct((NAP,), jnp.int32),
            jax.ShapeDtypeStruct((P,), jnp.int32),
        ],
        scratch_types=[
            pltpu.VMEM((NA,), jnp.int32),
            pltpu.VMEM((E,), jnp.int32),
            pltpu.VMEM((E,), jnp.int32),
            pltpu.VMEM((E,), jnp.int32),
            pltpu.VMEM((E,), jnp.int32),
            pltpu.VMEM((48,), jnp.int32),
            pltpu.VMEM((NAP,), jnp.int32),
            pltpu.VMEM((P,), jnp.int32),
            pltpu.VMEM((NB,), jnp.int32),
            pltpu.VMEM((NA,), jnp.int32),
            pltpu.VMEM((NA,), jnp.int32),
        ],
    )(_sc_meta_body)
    return k(eid_flat)


def kernel(x, expert_weights, expert_indices, scores, W1, W2):
    x2 = x.reshape(SEQ, D)
    x_d = x2[:TRIM]
    x_s = x2[TRIM:]
    w_s = expert_weights.reshape(SEQ, TOPK)[TRIM:]
    eid = expert_indices.reshape(SEQ, TOPK)[TRIM:].astype(jnp.int32)
    sc_d = scores[:TRIM]

    block_expert, invperm, gather_tok = _sc_metadata(eid.reshape(-1))

    W1b = W1.astype(jnp.bfloat16)
    W2b = W2.astype(jnp.bfloat16)
    xs_i32 = _bf16_to_i32(x_s.astype(jnp.bfloat16))
    xg = _i32_to_bf16(_sc_row_gather(xs_i32, gather_tok, 128))
    out_sorted = pl.pallas_call(
        _grouped_mlp_body,
        grid_spec=pltpu.PrefetchScalarGridSpec(
            num_scalar_prefetch=1,
            grid=(NB,),
            in_specs=[
                pl.BlockSpec((BLK, D), lambda b, s: (b, 0)),
                pl.BlockSpec((1, D, F), lambda b, s: (s[b], 0, 0)),
                pl.BlockSpec((1, F, D), lambda b, s: (s[b], 0, 0)),
            ],
            out_specs=pl.BlockSpec((BLK, D), lambda b, s: (b, 0)),
        ),
        out_shape=jax.ShapeDtypeStruct((P, D), jnp.bfloat16),
    )(block_expert, xg, W1b, W2b)

    rows = _i32_to_bf16(
        _sc_row_gather(_bf16_to_i32(out_sorted), invperm, 128)
    ).reshape(NAP // TOPK, TOPK, D)
    y_s, buffer = pl.pallas_call(
        _sparse_out_body,
        grid=(NTB,),
        in_specs=[
            pl.BlockSpec((TB, TOPK, D), lambda i: (i, 0, 0)),
            pl.BlockSpec((TB, TOPK), lambda i: (i, 0)),
            pl.BlockSpec((TB, TOPK), lambda i: (i, 0), memory_space=pltpu.SMEM),
        ],
        out_specs=[
            pl.BlockSpec((TB, D), lambda i: (i, 0)),
            pl.BlockSpec((TB, E, D), lambda i: (i + TRIM // TB, 0, 0)),
        ],
        out_shape=[
            jax.ShapeDtypeStruct((NS, D), jnp.float32),
            jax.ShapeDtypeStruct((SEQ, E, D), jnp.float32),
        ],
    )(rows, w_s, eid)

    y_d, buffer = pl.pallas_call(
        _dense_body,
        grid=(E // EG,),
        in_specs=[
            pl.BlockSpec((TRIM, D), lambda e: (0, 0)),
            pl.BlockSpec((EG, D, F), lambda e: (e, 0, 0)),
            pl.BlockSpec((EG, F, D), lambda e: (e, 0, 0)),
            pl.BlockSpec((TRIM, E), lambda e: (0, 0)),
            pl.BlockSpec(memory_space=pltpu.MemorySpace.HBM),
        ],
        out_specs=[
            pl.BlockSpec((TRIM, D), lambda e: (0, 0)),
            pl.BlockSpec((TRIM, EG, D), lambda e: (0, e, 0)),
        ],
        out_shape=[
            jax.ShapeDtypeStruct((TRIM, D), jnp.float32),
            jax.ShapeDtypeStruct((SEQ, E, D), jnp.float32),
        ],
        input_output_aliases={4: 1},
    )(x_d, W1b, W2b, sc_d, buffer)

    x_out = jnp.concatenate([y_d, y_s], axis=0).reshape(SEQ, 1, D)
    return (x_out, jnp.asarray(TRIM, jnp.int32), buffer.reshape(SEQ, E, D))


# speedup vs baseline: 1.0006x; 1.0006x over previous
import functools

import jax
import jax.numpy as jnp
from jax import lax
from jax.experimental import pallas as pl
from jax.experimental.pallas import tpu as pltpu
from jax.experimental.pallas import tpu_sc as plsc

E = 64
TOPK = 2
D = 768
F = 256
SEQ = 2048
TRIM = 32
NS = SEQ - TRIM
NA = NS * TOPK
BLK = 64
NB = 128
P = NB * BLK
TB = 8
NTB = NS // TB
NAP = 4096


def _grouped_mlp_body(be_ref, xg_ref, w1_ref, w2_ref, out_ref):
    h = jax.nn.gelu(
        jnp.dot(xg_ref[...], w1_ref[0], preferred_element_type=jnp.float32)
    )
    out_ref[...] = jnp.dot(h.astype(jnp.bfloat16), w2_ref[0],
                           preferred_element_type=jnp.float32
                           ).astype(jnp.bfloat16)


def _sparse_out_body(rows_ref, w_ref, eid_ref, _buf_in, y_ref, buf_ref):
    rows = rows_ref[...].astype(jnp.float32)
    w = w_ref[...]
    acc = rows[:, 0, :] * w[:, 0:1] + rows[:, 1, :] * w[:, 1:2]
    y_ref[...] = acc
    buf_ref[...] = jnp.zeros((TB, E, D), jnp.float32)
    for t in range(TB):
        e0 = eid_ref[t, 0]
        e1 = eid_ref[t, 1]
        dup = e0 == e1
        both = rows[t, 0, :] + rows[t, 1, :]
        v0 = jnp.where(dup, both, rows[t, 0, :])
        v1 = jnp.where(dup, both, rows[t, 1, :])
        buf_ref[t, e0, :] = v0
        buf_ref[t, e1, :] = v1


EG = 8


def _dense_body(xd_ref, w1_ref, w2_ref, sc_ref, y_ref, buf_ref):
    eg = pl.program_id(0)
    xd = xd_ref[...].astype(jnp.bfloat16)
    r = jax.lax.broadcasted_iota(jnp.int32, (E, EG), 0)
    c = jax.lax.broadcasted_iota(jnp.int32, (E, EG), 1)
    onehot = (r == EG * eg + c).astype(jnp.float32)
    w8 = jnp.dot(sc_ref[...], onehot, preferred_element_type=jnp.float32)

    outs = []
    acc = jnp.zeros((TRIM, D), jnp.float32)
    for j in range(EG):
        h = jax.nn.gelu(
            jnp.dot(xd, w1_ref[j], preferred_element_type=jnp.float32))
        out = jnp.dot(h.astype(jnp.bfloat16), w2_ref[j],
                      preferred_element_type=jnp.float32)
        outs.append(out)
        acc += out * w8[:, j:j + 1]
    buf_ref[...] = jnp.stack(outs, axis=1)

    @pl.when(eg == 0)
    def _():
        y_ref[...] = jnp.zeros((TRIM, D), jnp.float32)

    y_ref[...] += acc


def _exclusive_cumsum(v):
    return jnp.concatenate([jnp.zeros((1,), v.dtype), jnp.cumsum(v)[:-1]])


_NV = NA // 16
_LANE = None


def _sc_meta_body(eid_hbm, bexp_hbm, inv_hbm, gtok_hbm,
                  ev, hist, base, bstart, counters, tmp, inv_v, gtok_v,
                  bexp_v, occ_v, last_v):
    wid = lax.axis_index("s") * 2 + lax.axis_index("c")

    @pl.when(wid == 0)
    def _():
        iota = lax.iota(jnp.int32, 16)
        zeros16 = jnp.zeros((16,), jnp.int32)
        pltpu.sync_copy(eid_hbm, ev)

        def _z(i, _):
            gtok_v[pl.ds(i * 16, 16)] = zeros16
            return 0
        lax.fori_loop(0, P // 16, _z, 0)
        for vi in range(4):
            hist[pl.ds(vi * 16, 16)] = zeros16
            counters[pl.ds(vi * 16, 16)] = zeros16
        tmp[pl.ds(0, 16)] = zeros16
        tmp[pl.ds(16, 16)] = zeros16
        tmp[pl.ds(32, 16)] = zeros16
        for vi in range((NAP - NA) // 16):
            inv_v[pl.ds(NA + vi * 16, 16)] = zeros16

        def _hist_step(i, _):
            v = ev[pl.ds(i * 16, 16)]

            def _dup(m, carry):
                occ, cntl = carry
                sp = plsc.load_gather(ev, [jnp.zeros((16,), jnp.int32)
                                           + (i * 16 + m)])
                eq = v == sp
                occ = occ + jnp.where(eq & (iota > m), 1, 0)
                cntl = cntl + jnp.where(eq & (iota < m), 1, 0)
                return occ, cntl
            occ, cntl = lax.fori_loop(0, 16, _dup, (zeros16, zeros16))
            is_last = cntl == 0
            prior = plsc.load_gather(hist, [v])
            plsc.store_scatter(hist, [v], prior + occ + 1, mask=is_last)
            occ_v[pl.ds(i * 16, 16)] = occ
            last_v[pl.ds(i * 16, 16)] = jnp.where(is_last, 1, 0)
            return 0
        lax.fori_loop(0, _NV, _hist_step, 0)

        def _incl_scan(vec):
            cur = vec
            for sh in (1, 2, 4, 8):
                tmp[pl.ds(16, 16)] = cur
                cur = cur + tmp[pl.ds(16 - sh, 16)]
            tmp[pl.ds(32, 16)] = cur
            return cur

        carry_p = jnp.zeros((16,), jnp.int32)
        carry_b = jnp.zeros((16,), jnp.int32)
        full15 = jnp.full((16,), 15, jnp.int32)
        for vi in range(4):
            h = hist[pl.ds(vi * 16, 16)]
            nb = (h + 63) >> 6
            pad = nb * BLK
            incl_p = _incl_scan(pad)
            base[pl.ds(vi * 16, 16)] = incl_p - pad + carry_p
            carry_p = carry_p + plsc.load_gather(tmp, [full15 + 32])
            incl_b = _incl_scan(nb)
            bstart[pl.ds(vi * 16, 16)] = incl_b - nb + carry_b
            carry_b = carry_b + plsc.load_gather(tmp, [full15 + 32])

        for bi in range(NB // 16):
            bvec = bi * 16 + iota

            def _acc(e, a):
                bs_e = plsc.load_gather(bstart, [jnp.zeros((16,), jnp.int32)
                                                 + e])
                return a + jnp.where(bvec >= bs_e, 1, 0)
            bexp_v[pl.ds(bi * 16, 16)] = lax.fori_loop(
                0, E, _acc, jnp.full((16,), -1, jnp.int32))

        def _slot_step(i, _):
            v = ev[pl.ds(i * 16, 16)]
            occ = occ_v[pl.ds(i * 16, 16)]
            is_last = last_v[pl.ds(i * 16, 16)] == 1
            prior = plsc.load_gather(counters, [v])
            rank = prior + occ
            plsc.store_scatter(counters, [v], rank + 1, mask=is_last)
            slot = plsc.load_gather(base, [v]) + rank
            jg = i * 16 + iota
            inv_v[pl.ds(i * 16, 16)] = slot
            plsc.store_scatter(gtok_v, [slot], jg >> 1)
            return 0
        lax.fori_loop(0, _NV, _slot_step, 0)

        pltpu.sync_copy(bexp_v, bexp_hbm)
        pltpu.sync_copy(inv_v, inv_hbm)
        pltpu.sync_copy(gtok_v, gtok_hbm)


def _sc_row_gather(table_i32, idx, chunk):
    n_rows, d = idx.shape[0], table_i32.shape[1]
    nw = 32
    per_w = n_rows // nw
    nch = per_w // chunk
    mesh = plsc.VectorSubcoreMesh(core_axis_name="c", subcore_axis_name="s")

    def body(table_hbm, idx_hbm, out_hbm, idx_v, rows_v, sem):
        wid = lax.axis_index("s") * 2 + lax.axis_index("c")
        base = wid * per_w
        for c in range(nch):
            b = base + c * chunk
            pltpu.sync_copy(idx_hbm.at[pl.ds(b, chunk)], idx_v)
            pltpu.async_copy(table_hbm.at[idx_v], rows_v, sem).wait()
            pltpu.sync_copy(rows_v, out_hbm.at[pl.ds(b, chunk)])

    k = functools.partial(
        pl.kernel, mesh=mesh,
        out_type=jax.ShapeDtypeStruct((n_rows, d), jnp.int32),
        scratch_types=[
            pltpu.VMEM((chunk,), jnp.int32),
            pltpu.VMEM((chunk, d), jnp.int32),
            pltpu.SemaphoreType.DMA,
        ],
    )(body)
    return k(table_i32, idx)


def _bf16_to_i32(x):
    return lax.bitcast_convert_type(
        x.reshape(*x.shape[:-1], x.shape[-1] // 2, 2), jnp.int32)


def _i32_to_bf16(x):
    y = lax.bitcast_convert_type(x, jnp.bfloat16)
    return y.reshape(*x.shape[:-1], x.shape[-1] * 2)


def _sc_metadata(eid_flat):
    mesh = plsc.VectorSubcoreMesh(core_axis_name="c", subcore_axis_name="s")
    k = functools.partial(
        pl.kernel, mesh=mesh,
        compiler_params=pltpu.CompilerParams(needs_layout_passes=False),
        out_type=[
            jax.ShapeDtypeStruct((NB,), jnp.int32),
            jax.ShapeDtypeStruct((NAP,), jnp.int32),
            jax.ShapeDtypeStruct((P,), jnp.int32),
        ],
        scratch_types=[
            pltpu.VMEM((NA,), jnp.int32),
            pltpu.VMEM((E,), jnp.int32),
            pltpu.VMEM((E,), jnp.int32),
            pltpu.VMEM((E,), jnp.int32),
            pltpu.VMEM((E,), jnp.int32),
            pltpu.VMEM((48,), jnp.int32),
            pltpu.VMEM((NAP,), jnp.int32),
            pltpu.VMEM((P,), jnp.int32),
            pltpu.VMEM((NB,), jnp.int32),
            pltpu.VMEM((NA,), jnp.int32),
            pltpu.VMEM((NA,), jnp.int32),
        ],
    )(_sc_meta_body)
    return k(eid_flat)


def kernel(x, expert_weights, expert_indices, scores, W1, W2):
    x2 = x.reshape(SEQ, D)
    x_d = x2[:TRIM]
    x_s = x2[TRIM:]
    w_s = expert_weights.reshape(SEQ, TOPK)[TRIM:]
    eid = expert_indices.reshape(SEQ, TOPK)[TRIM:].astype(jnp.int32)
    sc_d = scores[:TRIM]

    block_expert, invperm, gather_tok = _sc_metadata(eid.reshape(-1))

    W1b = W1.astype(jnp.bfloat16)
    W2b = W2.astype(jnp.bfloat16)
    xs_i32 = _bf16_to_i32(x_s.astype(jnp.bfloat16))
    xg = _i32_to_bf16(_sc_row_gather(xs_i32, gather_tok, 128))
    out_sorted = pl.pallas_call(
        _grouped_mlp_body,
        grid_spec=pltpu.PrefetchScalarGridSpec(
            num_scalar_prefetch=1,
            grid=(NB,),
            in_specs=[
                pl.BlockSpec((BLK, D), lambda b, s: (b, 0)),
                pl.BlockSpec((1, D, F), lambda b, s: (s[b], 0, 0)),
                pl.BlockSpec((1, F, D), lambda b, s: (s[b], 0, 0)),
            ],
            out_specs=pl.BlockSpec((BLK, D), lambda b, s: (b, 0)),
        ),
        out_shape=jax.ShapeDtypeStruct((P, D), jnp.bfloat16),
    )(block_expert, xg, W1b, W2b)

    y_d, buffer = pl.pallas_call(
        _dense_body,
        grid=(E // EG,),
        in_specs=[
            pl.BlockSpec((TRIM, D), lambda e: (0, 0)),
            pl.BlockSpec((EG, D, F), lambda e: (e, 0, 0)),
            pl.BlockSpec((EG, F, D), lambda e: (e, 0, 0)),
            pl.BlockSpec((TRIM, E), lambda e: (0, 0)),
        ],
        out_specs=[
            pl.BlockSpec((TRIM, D), lambda e: (0, 0)),
            pl.BlockSpec((TRIM, EG, D), lambda e: (0, e, 0)),
        ],
        out_shape=[
            jax.ShapeDtypeStruct((TRIM, D), jnp.float32),
            jax.ShapeDtypeStruct((SEQ, E, D), jnp.float32),
        ],
    )(x_d, W1b, W2b, sc_d)

    rows = _i32_to_bf16(
        _sc_row_gather(_bf16_to_i32(out_sorted), invperm, 128)
    ).reshape(NAP // TOPK, TOPK, D)
    y_s, buffer = pl.pallas_call(
        _sparse_out_body,
        grid=(NTB,),
        in_specs=[
            pl.BlockSpec((TB, TOPK, D), lambda i: (i, 0, 0)),
            pl.BlockSpec((TB, TOPK), lambda i: (i, 0)),
            pl.BlockSpec((TB, TOPK), lambda i: (i, 0), memory_space=pltpu.SMEM),
            pl.BlockSpec(memory_space=pltpu.MemorySpace.HBM),
        ],
        out_specs=[
            pl.BlockSpec((TB, D), lambda i: (i, 0)),
            pl.BlockSpec((TB, E, D), lambda i: (i + TRIM // TB, 0, 0)),
        ],
        out_shape=[
            jax.ShapeDtypeStruct((NS, D), jnp.float32),
            jax.ShapeDtypeStruct((SEQ, E, D), jnp.float32),
        ],
        input_output_aliases={3: 1},
    )(rows, w_s, eid, buffer)

    x_out = jnp.concatenate([y_d, y_s], axis=0).reshape(SEQ, 1, D)
    return (x_out, jnp.asarray(TRIM, jnp.int32), buffer.reshape(SEQ, E, D))


# speedup vs baseline: 2.3058x; 2.3045x over previous
import functools

import jax
import jax.numpy as jnp
from jax import lax
from jax.experimental import pallas as pl
from jax.experimental.pallas import tpu as pltpu
from jax.experimental.pallas import tpu_sc as plsc

E = 64
TOPK = 2
D = 768
F = 256
SEQ = 2048
TRIM = 32
NS = SEQ - TRIM
NA = NS * TOPK
BLK = 64
NB = 128
P = NB * BLK
TB = 8
NTB = NS // TB
NAP = 4096


def _grouped_mlp_body(be_ref, xg_ref, w1_ref, w2_ref, out_ref):
    h = jax.nn.gelu(
        jnp.dot(xg_ref[...].astype(jnp.bfloat16), w1_ref[0],
                preferred_element_type=jnp.float32)
    )
    out_ref[...] = jnp.dot(h.astype(jnp.bfloat16), w2_ref[0],
                           preferred_element_type=jnp.float32)


def _sparse_out_body(rows_ref, w_ref, eid_ref, _buf_in, y_ref, buf_ref):
    rows = rows_ref[...].astype(jnp.float32)
    w = w_ref[...]
    acc = rows[:, 0, :] * w[:, 0:1] + rows[:, 1, :] * w[:, 1:2]
    y_ref[...] = acc
    buf_ref[...] = jnp.zeros((TB, E, D), jnp.float32)
    for t in range(TB):
        e0 = eid_ref[t, 0]
        e1 = eid_ref[t, 1]
        dup = e0 == e1
        both = rows[t, 0, :] + rows[t, 1, :]
        v0 = jnp.where(dup, both, rows[t, 0, :])
        v1 = jnp.where(dup, both, rows[t, 1, :])
        buf_ref[t, e0, :] = v0
        buf_ref[t, e1, :] = v1


EG = 8


def _dense_body(xd_ref, w1_ref, w2_ref, sc_ref, y_ref, buf_ref):
    eg = pl.program_id(0)
    xd = xd_ref[...].astype(jnp.bfloat16)
    r = jax.lax.broadcasted_iota(jnp.int32, (E, EG), 0)
    c = jax.lax.broadcasted_iota(jnp.int32, (E, EG), 1)
    onehot = (r == EG * eg + c).astype(jnp.float32)
    w8 = jnp.dot(sc_ref[...], onehot, preferred_element_type=jnp.float32)

    outs = []
    acc = jnp.zeros((TRIM, D), jnp.float32)
    for j in range(EG):
        h = jax.nn.gelu(
            jnp.dot(xd, w1_ref[j], preferred_element_type=jnp.float32))
        out = jnp.dot(h.astype(jnp.bfloat16), w2_ref[j],
                      preferred_element_type=jnp.float32)
        outs.append(out)
        acc += out * w8[:, j:j + 1]
    buf_ref[...] = jnp.stack(outs, axis=1)

    @pl.when(eg == 0)
    def _():
        y_ref[...] = jnp.zeros((TRIM, D), jnp.float32)

    y_ref[...] += acc


def _exclusive_cumsum(v):
    return jnp.concatenate([jnp.zeros((1,), v.dtype), jnp.cumsum(v)[:-1]])


_NV = NA // 16
_LANE = None


def _sc_meta_body(eid_hbm, bexp_hbm, inv_hbm, gtok_hbm,
                  ev, hist, base, bstart, counters, tmp, inv_v, gtok_v,
                  bexp_v, occ_v, last_v):
    wid = lax.axis_index("s") * 2 + lax.axis_index("c")

    @pl.when(wid == 0)
    def _():
        iota = lax.iota(jnp.int32, 16)
        zeros16 = jnp.zeros((16,), jnp.int32)
        pltpu.sync_copy(eid_hbm, ev)

        def _z(i, _):
            gtok_v[pl.ds(i * 16, 16)] = zeros16
            return 0
        lax.fori_loop(0, P // 16, _z, 0)
        for vi in range(4):
            hist[pl.ds(vi * 16, 16)] = zeros16
            counters[pl.ds(vi * 16, 16)] = zeros16
        tmp[pl.ds(0, 16)] = zeros16
        tmp[pl.ds(16, 16)] = zeros16
        tmp[pl.ds(32, 16)] = zeros16
        for vi in range((NAP - NA) // 16):
            inv_v[pl.ds(NA + vi * 16, 16)] = zeros16

        def _hist_step(i, _):
            v = ev[pl.ds(i * 16, 16)]

            def _dup(m, carry):
                occ, cntl = carry
                sp = plsc.load_gather(ev, [jnp.zeros((16,), jnp.int32)
                                           + (i * 16 + m)])
                eq = v == sp
                occ = occ + jnp.where(eq & (iota > m), 1, 0)
                cntl = cntl + jnp.where(eq & (iota < m), 1, 0)
                return occ, cntl
            occ, cntl = lax.fori_loop(0, 16, _dup, (zeros16, zeros16))
            is_last = cntl == 0
            prior = plsc.load_gather(hist, [v])
            plsc.store_scatter(hist, [v], prior + occ + 1, mask=is_last)
            occ_v[pl.ds(i * 16, 16)] = occ
            last_v[pl.ds(i * 16, 16)] = jnp.where(is_last, 1, 0)
            return 0
        lax.fori_loop(0, _NV, _hist_step, 0)

        def _incl_scan(vec):
            cur = vec
            for sh in (1, 2, 4, 8):
                tmp[pl.ds(16, 16)] = cur
                cur = cur + tmp[pl.ds(16 - sh, 16)]
            tmp[pl.ds(32, 16)] = cur
            return cur

        carry_p = jnp.zeros((16,), jnp.int32)
        carry_b = jnp.zeros((16,), jnp.int32)
        full15 = jnp.full((16,), 15, jnp.int32)
        for vi in range(4):
            h = hist[pl.ds(vi * 16, 16)]
            nb = (h + 63) >> 6
            pad = nb * BLK
            incl_p = _incl_scan(pad)
            base[pl.ds(vi * 16, 16)] = incl_p - pad + carry_p
            carry_p = carry_p + plsc.load_gather(tmp, [full15 + 32])
            incl_b = _incl_scan(nb)
            bstart[pl.ds(vi * 16, 16)] = incl_b - nb + carry_b
            carry_b = carry_b + plsc.load_gather(tmp, [full15 + 32])

        for bi in range(NB // 16):
            bvec = bi * 16 + iota

            def _acc(e, a):
                bs_e = plsc.load_gather(bstart, [jnp.zeros((16,), jnp.int32)
                                                 + e])
                return a + jnp.where(bvec >= bs_e, 1, 0)
            bexp_v[pl.ds(bi * 16, 16)] = lax.fori_loop(
                0, E, _acc, jnp.full((16,), -1, jnp.int32))

        def _slot_step(i, _):
            v = ev[pl.ds(i * 16, 16)]
            occ = occ_v[pl.ds(i * 16, 16)]
            is_last = last_v[pl.ds(i * 16, 16)] == 1
            prior = plsc.load_gather(counters, [v])
            rank = prior + occ
            plsc.store_scatter(counters, [v], rank + 1, mask=is_last)
            slot = plsc.load_gather(base, [v]) + rank
            jg = i * 16 + iota
            inv_v[pl.ds(i * 16, 16)] = slot
            plsc.store_scatter(gtok_v, [slot], jg >> 1)
            return 0
        lax.fori_loop(0, _NV, _slot_step, 0)

        pltpu.sync_copy(bexp_v, bexp_hbm)
        pltpu.sync_copy(inv_v, inv_hbm)
        pltpu.sync_copy(gtok_v, gtok_hbm)


def _sc_row_gather(table, idx, chunk):
    n_rows, d = idx.shape[0], table.shape[1]
    dt = table.dtype
    nw = 32
    per_w = n_rows // nw
    nch = per_w // chunk
    mesh = plsc.VectorSubcoreMesh(core_axis_name="c", subcore_axis_name="s")

    def body(table_hbm, idx_hbm, out_hbm, idx_v, rows_v, sem):
        wid = lax.axis_index("s") * 2 + lax.axis_index("c")
        base = wid * per_w
        for c in range(nch):
            b = base + c * chunk
            pltpu.sync_copy(idx_hbm.at[pl.ds(b, chunk)], idx_v)
            pltpu.async_copy(table_hbm.at[idx_v], rows_v, sem).wait()
            pltpu.sync_copy(rows_v, out_hbm.at[pl.ds(b, chunk)])

    k = functools.partial(
        pl.kernel, mesh=mesh,
        out_type=jax.ShapeDtypeStruct((n_rows, d), dt),
        scratch_types=[
            pltpu.VMEM((chunk,), jnp.int32),
            pltpu.VMEM((chunk, d), dt),
            pltpu.SemaphoreType.DMA,
        ],
    )(body)
    return k(table, idx)


def _sc_metadata(eid_flat):
    mesh = plsc.VectorSubcoreMesh(core_axis_name="c", subcore_axis_name="s")
    k = functools.partial(
        pl.kernel, mesh=mesh,
        compiler_params=pltpu.CompilerParams(needs_layout_passes=False),
        out_type=[
            jax.ShapeDtypeStruct((NB,), jnp.int32),
            jax.ShapeDtypeStruct((NAP,), jnp.int32),
            jax.ShapeDtypeStruct((P,), jnp.int32),
        ],
        scratch_types=[
            pltpu.VMEM((NA,), jnp.int32),
            pltpu.VMEM((E,), jnp.int32),
            pltpu.VMEM((E,), jnp.int32),
            pltpu.VMEM((E,), jnp.int32),
            pltpu.VMEM((E,), jnp.int32),
            pltpu.VMEM((48,), jnp.int32),
            pltpu.VMEM((NAP,), jnp.int32),
            pltpu.VMEM((P,), jnp.int32),
            pltpu.VMEM((NB,), jnp.int32),
            pltpu.VMEM((NA,), jnp.int32),
            pltpu.VMEM((NA,), jnp.int32),
        ],
    )(_sc_meta_body)
    return k(eid_flat)


def kernel(x, expert_weights, expert_indices, scores, W1, W2):
    x2 = x.reshape(SEQ, D)
    x_d = x2[:TRIM]
    x_s = x2[TRIM:]
    w_s = expert_weights.reshape(SEQ, TOPK)[TRIM:]
    eid = expert_indices.reshape(SEQ, TOPK)[TRIM:].astype(jnp.int32)
    sc_d = scores[:TRIM]

    block_expert, invperm, gather_tok = _sc_metadata(eid.reshape(-1))

    W1b = W1.astype(jnp.bfloat16)
    W2b = W2.astype(jnp.bfloat16)
    xg = _sc_row_gather(x_s, gather_tok, 128)
    out_sorted = pl.pallas_call(
        _grouped_mlp_body,
        grid_spec=pltpu.PrefetchScalarGridSpec(
            num_scalar_prefetch=1,
            grid=(NB,),
            in_specs=[
                pl.BlockSpec((BLK, D), lambda b, s: (b, 0)),
                pl.BlockSpec((1, D, F), lambda b, s: (s[b], 0, 0)),
                pl.BlockSpec((1, F, D), lambda b, s: (s[b], 0, 0)),
            ],
            out_specs=pl.BlockSpec((BLK, D), lambda b, s: (b, 0)),
        ),
        out_shape=jax.ShapeDtypeStruct((P, D), jnp.float32),
    )(block_expert, xg, W1b, W2b)

    y_d, buffer = pl.pallas_call(
        _dense_body,
        grid=(E // EG,),
        in_specs=[
            pl.BlockSpec((TRIM, D), lambda e: (0, 0)),
            pl.BlockSpec((EG, D, F), lambda e: (e, 0, 0)),
            pl.BlockSpec((EG, F, D), lambda e: (e, 0, 0)),
            pl.BlockSpec((TRIM, E), lambda e: (0, 0)),
        ],
        out_specs=[
            pl.BlockSpec((TRIM, D), lambda e: (0, 0)),
            pl.BlockSpec((TRIM, EG, D), lambda e: (0, e, 0)),
        ],
        out_shape=[
            jax.ShapeDtypeStruct((TRIM, D), jnp.float32),
            jax.ShapeDtypeStruct((SEQ, E, D), jnp.float32),
        ],
    )(x_d, W1b, W2b, sc_d)

    rows = _sc_row_gather(out_sorted, invperm, 128).reshape(
        NAP // TOPK, TOPK, D)
    y_s, buffer = pl.pallas_call(
        _sparse_out_body,
        grid=(NTB,),
        in_specs=[
            pl.BlockSpec((TB, TOPK, D), lambda i: (i, 0, 0)),
            pl.BlockSpec((TB, TOPK), lambda i: (i, 0)),
            pl.BlockSpec((TB, TOPK), lambda i: (i, 0), memory_space=pltpu.SMEM),
            pl.BlockSpec(memory_space=pltpu.MemorySpace.HBM),
        ],
        out_specs=[
            pl.BlockSpec((TB, D), lambda i: (i, 0)),
            pl.BlockSpec((TB, E, D), lambda i: (i + TRIM // TB, 0, 0)),
        ],
        out_shape=[
            jax.ShapeDtypeStruct((NS, D), jnp.float32),
            jax.ShapeDtypeStruct((SEQ, E, D), jnp.float32),
        ],
        input_output_aliases={3: 1},
    )(rows, w_s, eid, buffer)

    x_out = jnp.concatenate([y_d, y_s], axis=0).reshape(SEQ, 1, D)
    return (x_out, jnp.asarray(TRIM, jnp.int32), buffer.reshape(SEQ, E, D))


# speedup vs baseline: 3.1504x; 1.3663x over previous
import functools

import jax
import jax.numpy as jnp
from jax import lax
from jax.experimental import pallas as pl
from jax.experimental.pallas import tpu as pltpu
from jax.experimental.pallas import tpu_sc as plsc

E = 64
TOPK = 2
D = 768
F = 256
SEQ = 2048
TRIM = 32
NS = SEQ - TRIM
NA = NS * TOPK
BLK = 64
NB = 128
P = NB * BLK
TB = 8
NTB = NS // TB
NAP = 4096


def _grouped_mlp_body(be_ref, xg_ref, w1_ref, w2_ref, out_ref):
    h = jax.nn.gelu(
        jnp.dot(xg_ref[...].astype(jnp.bfloat16), w1_ref[0],
                preferred_element_type=jnp.float32)
    )
    out_ref[...] = jnp.dot(h.astype(jnp.bfloat16), w2_ref[0],
                           preferred_element_type=jnp.float32)


def _sparse_out_body(rows_ref, w_ref, eid_ref, _buf_in, y_ref, buf_ref):
    rows = rows_ref[...].astype(jnp.float32)
    w = w_ref[...]
    acc = rows[:, 0, :] * w[:, 0:1] + rows[:, 1, :] * w[:, 1:2]
    y_ref[...] = acc
    buf_ref[...] = jnp.zeros((TB, E, D), jnp.float32)
    for t in range(TB):
        e0 = eid_ref[t, 0]
        e1 = eid_ref[t, 1]
        dup = e0 == e1
        both = rows[t, 0, :] + rows[t, 1, :]
        v0 = jnp.where(dup, both, rows[t, 0, :])
        v1 = jnp.where(dup, both, rows[t, 1, :])
        buf_ref[t, e0, :] = v0
        buf_ref[t, e1, :] = v1


EG = 8


def _dense_body(xd_ref, w1_ref, w2_ref, sc_ref, y_ref, buf_ref):
    eg = pl.program_id(0)
    xd = xd_ref[...].astype(jnp.bfloat16)
    r = jax.lax.broadcasted_iota(jnp.int32, (E, EG), 0)
    c = jax.lax.broadcasted_iota(jnp.int32, (E, EG), 1)
    onehot = (r == EG * eg + c).astype(jnp.float32)
    w8 = jnp.dot(sc_ref[...], onehot, preferred_element_type=jnp.float32)

    outs = []
    acc = jnp.zeros((TRIM, D), jnp.float32)
    for j in range(EG):
        h = jax.nn.gelu(
            jnp.dot(xd, w1_ref[j], preferred_element_type=jnp.float32))
        out = jnp.dot(h.astype(jnp.bfloat16), w2_ref[j],
                      preferred_element_type=jnp.float32)
        outs.append(out)
        acc += out * w8[:, j:j + 1]
    buf_ref[...] = jnp.stack(outs, axis=1)

    @pl.when(eg == 0)
    def _():
        y_ref[...] = jnp.zeros((TRIM, D), jnp.float32)

    y_ref[...] += acc


def _exclusive_cumsum(v):
    return jnp.concatenate([jnp.zeros((1,), v.dtype), jnp.cumsum(v)[:-1]])


_NV = NA // 16
_LANE = None


def _sc_meta_body(eid_hbm, bexp_hbm, inv_hbm, gtok_hbm,
                  ev, hist, base, bstart, counters, tmp, inv_v, gtok_v,
                  bexp_v, occ_v, last_v):
    wid = lax.axis_index("s") * 2 + lax.axis_index("c")

    @pl.when(wid == 0)
    def _():
        iota = lax.iota(jnp.int32, 16)
        zeros16 = jnp.zeros((16,), jnp.int32)
        pltpu.sync_copy(eid_hbm, ev)

        def _z(i, _):
            gtok_v[pl.ds(i * 16, 16)] = (i * 16 + iota) & 1023
            return 0
        lax.fori_loop(0, P // 16, _z, 0)
        for vi in range(4):
            hist[pl.ds(vi * 16, 16)] = zeros16
            counters[pl.ds(vi * 16, 16)] = zeros16
        tmp[pl.ds(0, 16)] = zeros16
        tmp[pl.ds(16, 16)] = zeros16
        tmp[pl.ds(32, 16)] = zeros16
        for vi in range((NAP - NA) // 16):
            inv_v[pl.ds(NA + vi * 16, 16)] = zeros16

        def _hist_step(i, _):
            v = ev[pl.ds(i * 16, 16)]

            def _dup(m, carry):
                occ, cntl = carry
                sp = plsc.load_gather(ev, [jnp.zeros((16,), jnp.int32)
                                           + (i * 16 + m)])
                eq = v == sp
                occ = occ + jnp.where(eq & (iota > m), 1, 0)
                cntl = cntl + jnp.where(eq & (iota < m), 1, 0)
                return occ, cntl
            occ, cntl = lax.fori_loop(0, 16, _dup, (zeros16, zeros16))
            is_last = cntl == 0
            prior = plsc.load_gather(hist, [v])
            plsc.store_scatter(hist, [v], prior + occ + 1, mask=is_last)
            occ_v[pl.ds(i * 16, 16)] = occ
            last_v[pl.ds(i * 16, 16)] = jnp.where(is_last, 1, 0)
            return 0
        lax.fori_loop(0, _NV, _hist_step, 0)

        def _incl_scan(vec):
            cur = vec
            for sh in (1, 2, 4, 8):
                tmp[pl.ds(16, 16)] = cur
                cur = cur + tmp[pl.ds(16 - sh, 16)]
            tmp[pl.ds(32, 16)] = cur
            return cur

        carry_p = jnp.zeros((16,), jnp.int32)
        carry_b = jnp.zeros((16,), jnp.int32)
        full15 = jnp.full((16,), 15, jnp.int32)
        for vi in range(4):
            h = hist[pl.ds(vi * 16, 16)]
            nb = (h + 63) >> 6
            pad = nb * BLK
            incl_p = _incl_scan(pad)
            base[pl.ds(vi * 16, 16)] = incl_p - pad + carry_p
            carry_p = carry_p + plsc.load_gather(tmp, [full15 + 32])
            incl_b = _incl_scan(nb)
            bstart[pl.ds(vi * 16, 16)] = incl_b - nb + carry_b
            carry_b = carry_b + plsc.load_gather(tmp, [full15 + 32])

        for bi in range(NB // 16):
            bvec = bi * 16 + iota

            def _acc(e, a):
                bs_e = plsc.load_gather(bstart, [jnp.zeros((16,), jnp.int32)
                                                 + e])
                return a + jnp.where(bvec >= bs_e, 1, 0)
            bexp_v[pl.ds(bi * 16, 16)] = lax.fori_loop(
                0, E, _acc, jnp.full((16,), -1, jnp.int32))

        def _slot_step(i, _):
            v = ev[pl.ds(i * 16, 16)]
            occ = occ_v[pl.ds(i * 16, 16)]
            is_last = last_v[pl.ds(i * 16, 16)] == 1
            prior = plsc.load_gather(counters, [v])
            rank = prior + occ
            plsc.store_scatter(counters, [v], rank + 1, mask=is_last)
            slot = plsc.load_gather(base, [v]) + rank
            jg = i * 16 + iota
            inv_v[pl.ds(i * 16, 16)] = slot
            plsc.store_scatter(gtok_v, [slot], jg >> 1)
            return 0
        lax.fori_loop(0, _NV, _slot_step, 0)

        pltpu.sync_copy(bexp_v, bexp_hbm)
        pltpu.sync_copy(inv_v, inv_hbm)
        pltpu.sync_copy(gtok_v, gtok_hbm)


def _sc_row_gather(table, idx, chunk):
    n_rows, d = idx.shape[0], table.shape[1]
    dt = table.dtype
    nw = 32
    per_w = n_rows // nw
    nch = per_w // chunk
    mesh = plsc.VectorSubcoreMesh(core_axis_name="c", subcore_axis_name="s")

    def body(table_hbm, idx_hbm, out_hbm, idx_v, rows_v, sem):
        wid = lax.axis_index("s") * 2 + lax.axis_index("c")
        base = wid * per_w
        for c in range(nch):
            b = base + c * chunk
            pltpu.sync_copy(idx_hbm.at[pl.ds(b, chunk)], idx_v)
            pltpu.async_copy(table_hbm.at[idx_v], rows_v, sem).wait()
            pltpu.sync_copy(rows_v, out_hbm.at[pl.ds(b, chunk)])

    k = functools.partial(
        pl.kernel, mesh=mesh,
        out_type=jax.ShapeDtypeStruct((n_rows, d), dt),
        scratch_types=[
            pltpu.VMEM((chunk,), jnp.int32),
            pltpu.VMEM((chunk, d), dt),
            pltpu.SemaphoreType.DMA,
        ],
    )(body)
    return k(table, idx)


def _sc_metadata(eid_flat):
    mesh = plsc.VectorSubcoreMesh(core_axis_name="c", subcore_axis_name="s")
    k = functools.partial(
        pl.kernel, mesh=mesh,
        compiler_params=pltpu.CompilerParams(needs_layout_passes=False),
        out_type=[
            jax.ShapeDtypeStruct((NB,), jnp.int32),
            jax.ShapeDtypeStruct((NAP,), jnp.int32),
            jax.ShapeDtypeStruct((P,), jnp.int32),
        ],
        scratch_types=[
            pltpu.VMEM((NA,), jnp.int32),
            pltpu.VMEM((E,), jnp.int32),
            pltpu.VMEM((E,), jnp.int32),
            pltpu.VMEM((E,), jnp.int32),
            pltpu.VMEM((E,), jnp.int32),
            pltpu.VMEM((48,), jnp.int32),
            pltpu.VMEM((NAP,), jnp.int32),
            pltpu.VMEM((P,), jnp.int32),
            pltpu.VMEM((NB,), jnp.int32),
            pltpu.VMEM((NA,), jnp.int32),
            pltpu.VMEM((NA,), jnp.int32),
        ],
    )(_sc_meta_body)
    return k(eid_flat)


def kernel(x, expert_weights, expert_indices, scores, W1, W2):
    x2 = x.reshape(SEQ, D)
    x_d = x2[:TRIM]
    x_s = x2[TRIM:]
    w_s = expert_weights.reshape(SEQ, TOPK)[TRIM:]
    eid = expert_indices.reshape(SEQ, TOPK)[TRIM:].astype(jnp.int32)
    sc_d = scores[:TRIM]

    block_expert, invperm, gather_tok = _sc_metadata(eid.reshape(-1))

    W1b = W1.astype(jnp.bfloat16)
    W2b = W2.astype(jnp.bfloat16)
    xg = _sc_row_gather(x_s, gather_tok, 128)
    out_sorted = pl.pallas_call(
        _grouped_mlp_body,
        grid_spec=pltpu.PrefetchScalarGridSpec(
            num_scalar_prefetch=1,
            grid=(NB,),
            in_specs=[
                pl.BlockSpec((BLK, D), lambda b, s: (b, 0)),
                pl.BlockSpec((1, D, F), lambda b, s: (s[b], 0, 0)),
                pl.BlockSpec((1, F, D), lambda b, s: (s[b], 0, 0)),
            ],
            out_specs=pl.BlockSpec((BLK, D), lambda b, s: (b, 0)),
        ),
        out_shape=jax.ShapeDtypeStruct((P, D), jnp.float32),
    )(block_expert, xg, W1b, W2b)

    y_d, buffer = pl.pallas_call(
        _dense_body,
        grid=(E // EG,),
        in_specs=[
            pl.BlockSpec((TRIM, D), lambda e: (0, 0)),
            pl.BlockSpec((EG, D, F), lambda e: (e, 0, 0)),
            pl.BlockSpec((EG, F, D), lambda e: (e, 0, 0)),
            pl.BlockSpec((TRIM, E), lambda e: (0, 0)),
        ],
        out_specs=[
            pl.BlockSpec((TRIM, D), lambda e: (0, 0)),
            pl.BlockSpec((TRIM, EG, D), lambda e: (0, e, 0)),
        ],
        out_shape=[
            jax.ShapeDtypeStruct((TRIM, D), jnp.float32),
            jax.ShapeDtypeStruct((SEQ, E, D), jnp.float32),
        ],
    )(x_d, W1b, W2b, sc_d)

    rows = _sc_row_gather(out_sorted, invperm, 128).reshape(
        NAP // TOPK, TOPK, D)
    y_s, buffer = pl.pallas_call(
        _sparse_out_body,
        grid=(NTB,),
        in_specs=[
            pl.BlockSpec((TB, TOPK, D), lambda i: (i, 0, 0)),
            pl.BlockSpec((TB, TOPK), lambda i: (i, 0)),
            pl.BlockSpec((TB, TOPK), lambda i: (i, 0), memory_space=pltpu.SMEM),
            pl.BlockSpec(memory_space=pltpu.MemorySpace.HBM),
        ],
        out_specs=[
            pl.BlockSpec((TB, D), lambda i: (i, 0)),
            pl.BlockSpec((TB, E, D), lambda i: (i + TRIM // TB, 0, 0)),
        ],
        out_shape=[
            jax.ShapeDtypeStruct((NS, D), jnp.float32),
            jax.ShapeDtypeStruct((SEQ, E, D), jnp.float32),
        ],
        input_output_aliases={3: 1},
    )(rows, w_s, eid, buffer)

    x_out = jnp.concatenate([y_d, y_s], axis=0).reshape(SEQ, 1, D)
    return (x_out, jnp.asarray(TRIM, jnp.int32), buffer.reshape(SEQ, E, D))


# speedup vs baseline: 3.1876x; 1.0118x over previous
import functools

import jax
import jax.numpy as jnp
from jax import lax
from jax.experimental import pallas as pl
from jax.experimental.pallas import tpu as pltpu
from jax.experimental.pallas import tpu_sc as plsc

E = 64
TOPK = 2
D = 768
F = 256
SEQ = 2048
TRIM = 32
NS = SEQ - TRIM
NA = NS * TOPK
BLK = 64
NB = 128
P = NB * BLK
TB = 8
NTB = NS // TB
NAP = 4096


def _grouped_mlp_body(be_ref, xg_ref, w1_ref, w2_ref, out_ref):
    @pl.when(pl.program_id(0) < be_ref[NB])
    def _():
        h = jax.nn.gelu(
            jnp.dot(xg_ref[...].astype(jnp.bfloat16), w1_ref[0],
                    preferred_element_type=jnp.float32)
        )
        out_ref[...] = jnp.dot(h.astype(jnp.bfloat16), w2_ref[0],
                               preferred_element_type=jnp.float32)


def _sparse_out_body(rows_ref, w_ref, eid_ref, _buf_in, y_ref, buf_ref):
    rows = rows_ref[...].astype(jnp.float32)
    w = w_ref[...]
    acc = rows[:, 0, :] * w[:, 0:1] + rows[:, 1, :] * w[:, 1:2]
    y_ref[...] = acc
    buf_ref[...] = jnp.zeros((TB, E, D), jnp.float32)
    for t in range(TB):
        e0 = eid_ref[t, 0]
        e1 = eid_ref[t, 1]
        dup = e0 == e1
        both = rows[t, 0, :] + rows[t, 1, :]
        v0 = jnp.where(dup, both, rows[t, 0, :])
        v1 = jnp.where(dup, both, rows[t, 1, :])
        buf_ref[t, e0, :] = v0
        buf_ref[t, e1, :] = v1


EG = 8


def _dense_body(xd_ref, w1_ref, w2_ref, sc_ref, y_ref, buf_ref):
    eg = pl.program_id(0)
    xd = xd_ref[...].astype(jnp.bfloat16)
    r = jax.lax.broadcasted_iota(jnp.int32, (E, EG), 0)
    c = jax.lax.broadcasted_iota(jnp.int32, (E, EG), 1)
    onehot = (r == EG * eg + c).astype(jnp.float32)
    w8 = jnp.dot(sc_ref[...], onehot, preferred_element_type=jnp.float32)

    outs = []
    acc = jnp.zeros((TRIM, D), jnp.float32)
    for j in range(EG):
        h = jax.nn.gelu(
            jnp.dot(xd, w1_ref[j], preferred_element_type=jnp.float32))
        out = jnp.dot(h.astype(jnp.bfloat16), w2_ref[j],
                      preferred_element_type=jnp.float32)
        outs.append(out)
        acc += out * w8[:, j:j + 1]
    buf_ref[...] = jnp.stack(outs, axis=1)

    @pl.when(eg == 0)
    def _():
        y_ref[...] = jnp.zeros((TRIM, D), jnp.float32)

    y_ref[...] += acc


def _exclusive_cumsum(v):
    return jnp.concatenate([jnp.zeros((1,), v.dtype), jnp.cumsum(v)[:-1]])


_NV = NA // 16
_LANE = None


def _sc_meta_body(eid_hbm, bexp_hbm, inv_hbm, gtok_hbm,
                  ev, hist, base, bstart, counters, tmp, inv_v, gtok_v,
                  bexp_v, occ_v, last_v):
    wid = lax.axis_index("s") * 2 + lax.axis_index("c")

    @pl.when(wid == 0)
    def _():
        iota = lax.iota(jnp.int32, 16)
        zeros16 = jnp.zeros((16,), jnp.int32)
        pltpu.sync_copy(eid_hbm, ev)

        def _z(i, _):
            gtok_v[pl.ds(i * 16, 16)] = (i * 16 + iota) & 1023
            return 0
        lax.fori_loop(0, P // 16, _z, 0)
        for vi in range(4):
            hist[pl.ds(vi * 16, 16)] = zeros16
            counters[pl.ds(vi * 16, 16)] = zeros16
        tmp[pl.ds(0, 16)] = zeros16
        tmp[pl.ds(16, 16)] = zeros16
        tmp[pl.ds(32, 16)] = zeros16
        for vi in range((NAP - NA) // 16):
            inv_v[pl.ds(NA + vi * 16, 16)] = zeros16

        def _hist_step(i, _):
            v = ev[pl.ds(i * 16, 16)]

            def _dup(m, carry):
                occ, cntl = carry
                sp = plsc.load_gather(ev, [jnp.zeros((16,), jnp.int32)
                                           + (i * 16 + m)])
                eq = v == sp
                occ = occ + jnp.where(eq & (iota > m), 1, 0)
                cntl = cntl + jnp.where(eq & (iota < m), 1, 0)
                return occ, cntl
            occ, cntl = lax.fori_loop(0, 16, _dup, (zeros16, zeros16))
            is_last = cntl == 0
            prior = plsc.load_gather(hist, [v])
            plsc.store_scatter(hist, [v], prior + occ + 1, mask=is_last)
            occ_v[pl.ds(i * 16, 16)] = occ
            last_v[pl.ds(i * 16, 16)] = jnp.where(is_last, 1, 0)
            return 0
        lax.fori_loop(0, _NV, _hist_step, 0)

        def _incl_scan(vec):
            cur = vec
            for sh in (1, 2, 4, 8):
                tmp[pl.ds(16, 16)] = cur
                cur = cur + tmp[pl.ds(16 - sh, 16)]
            tmp[pl.ds(32, 16)] = cur
            return cur

        carry_p = jnp.zeros((16,), jnp.int32)
        carry_b = jnp.zeros((16,), jnp.int32)
        full15 = jnp.full((16,), 15, jnp.int32)
        for vi in range(4):
            h = hist[pl.ds(vi * 16, 16)]
            nb = (h + 63) >> 6
            pad = nb * BLK
            incl_p = _incl_scan(pad)
            base[pl.ds(vi * 16, 16)] = incl_p - pad + carry_p
            carry_p = carry_p + plsc.load_gather(tmp, [full15 + 32])
            incl_b = _incl_scan(nb)
            bstart[pl.ds(vi * 16, 16)] = incl_b - nb + carry_b
            carry_b = carry_b + plsc.load_gather(tmp, [full15 + 32])

        for bi in range(NB // 16):
            bvec = bi * 16 + iota

            def _acc(e, a):
                bs_e = plsc.load_gather(bstart, [jnp.zeros((16,), jnp.int32)
                                                 + e])
                return a + jnp.where(bvec >= bs_e, 1, 0)
            bexp_v[pl.ds(bi * 16, 16)] = lax.fori_loop(
                0, E, _acc, jnp.full((16,), -1, jnp.int32))
        bexp_v[pl.ds(NB, 16)] = carry_b

        def _slot_step(i, _):
            v = ev[pl.ds(i * 16, 16)]
            occ = occ_v[pl.ds(i * 16, 16)]
            is_last = last_v[pl.ds(i * 16, 16)] == 1
            prior = plsc.load_gather(counters, [v])
            rank = prior + occ
            plsc.store_scatter(counters, [v], rank + 1, mask=is_last)
            slot = plsc.load_gather(base, [v]) + rank
            jg = i * 16 + iota
            inv_v[pl.ds(i * 16, 16)] = slot
            plsc.store_scatter(gtok_v, [slot], jg >> 1)
            return 0
        lax.fori_loop(0, _NV, _slot_step, 0)

        pltpu.sync_copy(bexp_v, bexp_hbm)
        pltpu.sync_copy(inv_v, inv_hbm)
        pltpu.sync_copy(gtok_v, gtok_hbm)


def _sc_row_gather(table, idx, chunk):
    n_rows, d = idx.shape[0], table.shape[1]
    dt = table.dtype
    nw = 32
    per_w = n_rows // nw
    nch = per_w // chunk
    mesh = plsc.VectorSubcoreMesh(core_axis_name="c", subcore_axis_name="s")

    def body(table_hbm, idx_hbm, out_hbm, idx_v, rows_v, sem):
        wid = lax.axis_index("s") * 2 + lax.axis_index("c")
        base = wid * per_w
        for c in range(nch):
            b = base + c * chunk
            pltpu.sync_copy(idx_hbm.at[pl.ds(b, chunk)], idx_v)
            pltpu.async_copy(table_hbm.at[idx_v], rows_v, sem).wait()
            pltpu.sync_copy(rows_v, out_hbm.at[pl.ds(b, chunk)])

    k = functools.partial(
        pl.kernel, mesh=mesh,
        out_type=jax.ShapeDtypeStruct((n_rows, d), dt),
        scratch_types=[
            pltpu.VMEM((chunk,), jnp.int32),
            pltpu.VMEM((chunk, d), dt),
            pltpu.SemaphoreType.DMA,
        ],
    )(body)
    return k(table, idx)


def _sc_metadata(eid_flat):
    mesh = plsc.VectorSubcoreMesh(core_axis_name="c", subcore_axis_name="s")
    k = functools.partial(
        pl.kernel, mesh=mesh,
        compiler_params=pltpu.CompilerParams(needs_layout_passes=False),
        out_type=[
            jax.ShapeDtypeStruct((NB + 16,), jnp.int32),
            jax.ShapeDtypeStruct((NAP,), jnp.int32),
            jax.ShapeDtypeStruct((P,), jnp.int32),
        ],
        scratch_types=[
            pltpu.VMEM((NA,), jnp.int32),
            pltpu.VMEM((E,), jnp.int32),
            pltpu.VMEM((E,), jnp.int32),
            pltpu.VMEM((E,), jnp.int32),
            pltpu.VMEM((E,), jnp.int32),
            pltpu.VMEM((48,), jnp.int32),
            pltpu.VMEM((NAP,), jnp.int32),
            pltpu.VMEM((P,), jnp.int32),
            pltpu.VMEM((NB + 16,), jnp.int32),
            pltpu.VMEM((NA,), jnp.int32),
            pltpu.VMEM((NA,), jnp.int32),
        ],
    )(_sc_meta_body)
    return k(eid_flat)


def kernel(x, expert_weights, expert_indices, scores, W1, W2):
    x2 = x.reshape(SEQ, D)
    x_d = x2[:TRIM]
    x_s = x2[TRIM:]
    w_s = expert_weights.reshape(SEQ, TOPK)[TRIM:]
    eid = expert_indices.reshape(SEQ, TOPK)[TRIM:].astype(jnp.int32)
    sc_d = scores[:TRIM]

    block_expert, invperm, gather_tok = _sc_metadata(eid.reshape(-1))

    W1b = W1.astype(jnp.bfloat16)
    W2b = W2.astype(jnp.bfloat16)
    xg = _sc_row_gather(x_s, gather_tok, 128)
    out_sorted = pl.pallas_call(
        _grouped_mlp_body,
        grid_spec=pltpu.PrefetchScalarGridSpec(
            num_scalar_prefetch=1,
            grid=(NB,),
            in_specs=[
                pl.BlockSpec((BLK, D), lambda b, s: (b, 0)),
                pl.BlockSpec((1, D, F), lambda b, s: (s[b], 0, 0)),
                pl.BlockSpec((1, F, D), lambda b, s: (s[b], 0, 0)),
            ],
            out_specs=pl.BlockSpec((BLK, D), lambda b, s: (b, 0)),
        ),
        out_shape=jax.ShapeDtypeStruct((P, D), jnp.float32),
    )(block_expert, xg, W1b, W2b)

    y_d, buffer = pl.pallas_call(
        _dense_body,
        grid=(E // EG,),
        in_specs=[
            pl.BlockSpec((TRIM, D), lambda e: (0, 0)),
            pl.BlockSpec((EG, D, F), lambda e: (e, 0, 0)),
            pl.BlockSpec((EG, F, D), lambda e: (e, 0, 0)),
            pl.BlockSpec((TRIM, E), lambda e: (0, 0)),
        ],
        out_specs=[
            pl.BlockSpec((TRIM, D), lambda e: (0, 0)),
            pl.BlockSpec((TRIM, EG, D), lambda e: (0, e, 0)),
        ],
        out_shape=[
            jax.ShapeDtypeStruct((TRIM, D), jnp.float32),
            jax.ShapeDtypeStruct((SEQ, E, D), jnp.float32),
        ],
    )(x_d, W1b, W2b, sc_d)

    rows = _sc_row_gather(out_sorted, invperm, 128).reshape(
        NAP // TOPK, TOPK, D)
    y_s, buffer = pl.pallas_call(
        _sparse_out_body,
        grid=(NTB,),
        in_specs=[
            pl.BlockSpec((TB, TOPK, D), lambda i: (i, 0, 0)),
            pl.BlockSpec((TB, TOPK), lambda i: (i, 0)),
            pl.BlockSpec((TB, TOPK), lambda i: (i, 0), memory_space=pltpu.SMEM),
            pl.BlockSpec(memory_space=pltpu.MemorySpace.HBM),
        ],
        out_specs=[
            pl.BlockSpec((TB, D), lambda i: (i, 0)),
            pl.BlockSpec((TB, E, D), lambda i: (i + TRIM // TB, 0, 0)),
        ],
        out_shape=[
            jax.ShapeDtypeStruct((NS, D), jnp.float32),
            jax.ShapeDtypeStruct((SEQ, E, D), jnp.float32),
        ],
        input_output_aliases={3: 1},
    )(rows, w_s, eid, buffer)

    x_out = jnp.concatenate([y_d, y_s], axis=0).reshape(SEQ, 1, D)
    return (x_out, jnp.asarray(TRIM, jnp.int32), buffer.reshape(SEQ, E, D))


# speedup vs baseline: 3.6041x; 1.1306x over previous
import functools

import jax
import jax.numpy as jnp
from jax import lax
from jax.experimental import pallas as pl
from jax.experimental.pallas import tpu as pltpu
from jax.experimental.pallas import tpu_sc as plsc

E = 64
TOPK = 2
D = 768
F = 256
SEQ = 2048
TRIM = 32
NS = SEQ - TRIM
NA = NS * TOPK
BLK = 64
NB = 128
P = NB * BLK
TB = 16
NTB = NS // TB
NAP = 4096


def _grouped_mlp_body(be_ref, xg_ref, w1_ref, w2_ref, out_ref):
    @pl.when(pl.program_id(0) < be_ref[NB])
    def _():
        h = jax.nn.gelu(
            jnp.dot(xg_ref[...].astype(jnp.bfloat16), w1_ref[0],
                    preferred_element_type=jnp.float32)
        )
        out_ref[...] = jnp.dot(h.astype(jnp.bfloat16), w2_ref[0],
                               preferred_element_type=jnp.float32)


def _sparse_out_body(rows_ref, w_ref, eid_ref, _buf_in, y_ref, buf_ref):
    rows = rows_ref[...].astype(jnp.float32)
    w = w_ref[...]
    acc = rows[:, 0, :] * w[:, 0:1] + rows[:, 1, :] * w[:, 1:2]
    y_ref[...] = acc
    buf_ref[...] = jnp.zeros((TB, E, D), jnp.float32)
    for t in range(TB):
        e0 = eid_ref[t, 0]
        e1 = eid_ref[t, 1]
        dup = e0 == e1
        both = rows[t, 0, :] + rows[t, 1, :]
        v0 = jnp.where(dup, both, rows[t, 0, :])
        v1 = jnp.where(dup, both, rows[t, 1, :])
        buf_ref[t, e0, :] = v0
        buf_ref[t, e1, :] = v1


EG = 8


def _dense_body(xd_ref, w1_ref, w2_ref, sc_ref, y_ref, buf_ref):
    eg = pl.program_id(0)
    xd = xd_ref[...].astype(jnp.bfloat16)
    r = jax.lax.broadcasted_iota(jnp.int32, (E, EG), 0)
    c = jax.lax.broadcasted_iota(jnp.int32, (E, EG), 1)
    onehot = (r == EG * eg + c).astype(jnp.float32)
    w8 = jnp.dot(sc_ref[...], onehot, preferred_element_type=jnp.float32)

    outs = []
    acc = jnp.zeros((TRIM, D), jnp.float32)
    for j in range(EG):
        h = jax.nn.gelu(
            jnp.dot(xd, w1_ref[j], preferred_element_type=jnp.float32))
        out = jnp.dot(h.astype(jnp.bfloat16), w2_ref[j],
                      preferred_element_type=jnp.float32)
        outs.append(out)
        acc += out * w8[:, j:j + 1]
    buf_ref[...] = jnp.stack(outs, axis=1)

    @pl.when(eg == 0)
    def _():
        y_ref[...] = jnp.zeros((TRIM, D), jnp.float32)

    y_ref[...] += acc


def _exclusive_cumsum(v):
    return jnp.concatenate([jnp.zeros((1,), v.dtype), jnp.cumsum(v)[:-1]])


_NV = NA // 16
_LANE = None


def _sc_meta_body(eid_hbm, bexp_hbm, inv_hbm, gtok_hbm,
                  ev, hist, base, bstart, counters, tmp, inv_v, gtok_v,
                  bexp_v, occ_v, last_v):
    wid = lax.axis_index("s") * 2 + lax.axis_index("c")

    @pl.when(wid == 0)
    def _():
        iota = lax.iota(jnp.int32, 16)
        zeros16 = jnp.zeros((16,), jnp.int32)
        pltpu.sync_copy(eid_hbm, ev)

        def _z(i, _):
            gtok_v[pl.ds(i * 16, 16)] = (i * 16 + iota) & 1023
            return 0
        lax.fori_loop(0, P // 16, _z, 0)
        for vi in range(4):
            hist[pl.ds(vi * 16, 16)] = zeros16
            counters[pl.ds(vi * 16, 16)] = zeros16
        tmp[pl.ds(0, 16)] = zeros16
        tmp[pl.ds(16, 16)] = zeros16
        tmp[pl.ds(32, 16)] = zeros16
        for vi in range((NAP - NA) // 16):
            inv_v[pl.ds(NA + vi * 16, 16)] = zeros16

        def _hist_step(i, _):
            v = ev[pl.ds(i * 16, 16)]

            def _dup(m, carry):
                occ, cntl = carry
                sp = plsc.load_gather(ev, [jnp.zeros((16,), jnp.int32)
                                           + (i * 16 + m)])
                eq = v == sp
                occ = occ + jnp.where(eq & (iota > m), 1, 0)
                cntl = cntl + jnp.where(eq & (iota < m), 1, 0)
                return occ, cntl
            occ, cntl = lax.fori_loop(0, 16, _dup, (zeros16, zeros16))
            is_last = cntl == 0
            prior = plsc.load_gather(hist, [v])
            plsc.store_scatter(hist, [v], prior + occ + 1, mask=is_last)
            occ_v[pl.ds(i * 16, 16)] = occ
            last_v[pl.ds(i * 16, 16)] = jnp.where(is_last, 1, 0)
            return 0
        lax.fori_loop(0, _NV, _hist_step, 0)

        def _incl_scan(vec):
            cur = vec
            for sh in (1, 2, 4, 8):
                tmp[pl.ds(16, 16)] = cur
                cur = cur + tmp[pl.ds(16 - sh, 16)]
            tmp[pl.ds(32, 16)] = cur
            return cur

        carry_p = jnp.zeros((16,), jnp.int32)
        carry_b = jnp.zeros((16,), jnp.int32)
        full15 = jnp.full((16,), 15, jnp.int32)
        for vi in range(4):
            h = hist[pl.ds(vi * 16, 16)]
            nb = (h + 63) >> 6
            pad = nb * BLK
            incl_p = _incl_scan(pad)
            base[pl.ds(vi * 16, 16)] = incl_p - pad + carry_p
            carry_p = carry_p + plsc.load_gather(tmp, [full15 + 32])
            incl_b = _incl_scan(nb)
            bstart[pl.ds(vi * 16, 16)] = incl_b - nb + carry_b
            carry_b = carry_b + plsc.load_gather(tmp, [full15 + 32])

        for bi in range(NB // 16):
            bvec = bi * 16 + iota

            def _acc(e, a):
                bs_e = plsc.load_gather(bstart, [jnp.zeros((16,), jnp.int32)
                                                 + e])
                return a + jnp.where(bvec >= bs_e, 1, 0)
            bexp_v[pl.ds(bi * 16, 16)] = lax.fori_loop(
                0, E, _acc, jnp.full((16,), -1, jnp.int32))
        bexp_v[pl.ds(NB, 16)] = carry_b

        def _slot_step(i, _):
            v = ev[pl.ds(i * 16, 16)]
            occ = occ_v[pl.ds(i * 16, 16)]
            is_last = last_v[pl.ds(i * 16, 16)] == 1
            prior = plsc.load_gather(counters, [v])
            rank = prior + occ
            plsc.store_scatter(counters, [v], rank + 1, mask=is_last)
            slot = plsc.load_gather(base, [v]) + rank
            jg = i * 16 + iota
            inv_v[pl.ds(i * 16, 16)] = slot
            plsc.store_scatter(gtok_v, [slot], jg >> 1)
            return 0
        lax.fori_loop(0, _NV, _slot_step, 0)

        pltpu.sync_copy(bexp_v, bexp_hbm)
        pltpu.sync_copy(inv_v, inv_hbm)
        pltpu.sync_copy(gtok_v, gtok_hbm)


def _sc_row_gather(table, idx, chunk):
    n_rows, d = idx.shape[0], table.shape[1]
    dt = table.dtype
    nw = 32
    per_w = n_rows // nw
    nch = per_w // chunk
    mesh = plsc.VectorSubcoreMesh(core_axis_name="c", subcore_axis_name="s")

    def body(table_hbm, idx_hbm, out_hbm, idx_v, rows_v, sem):
        wid = lax.axis_index("s") * 2 + lax.axis_index("c")
        base = wid * per_w
        for c in range(nch):
            b = base + c * chunk
            pltpu.sync_copy(idx_hbm.at[pl.ds(b, chunk)], idx_v)
            pltpu.async_copy(table_hbm.at[idx_v], rows_v, sem).wait()
            pltpu.sync_copy(rows_v, out_hbm.at[pl.ds(b, chunk)])

    k = functools.partial(
        pl.kernel, mesh=mesh,
        out_type=jax.ShapeDtypeStruct((n_rows, d), dt),
        scratch_types=[
            pltpu.VMEM((chunk,), jnp.int32),
            pltpu.VMEM((chunk, d), dt),
            pltpu.SemaphoreType.DMA,
        ],
    )(body)
    return k(table, idx)


def _sc_metadata(eid_flat):
    mesh = plsc.VectorSubcoreMesh(core_axis_name="c", subcore_axis_name="s")
    k = functools.partial(
        pl.kernel, mesh=mesh,
        compiler_params=pltpu.CompilerParams(needs_layout_passes=False),
        out_type=[
            jax.ShapeDtypeStruct((NB + 16,), jnp.int32),
            jax.ShapeDtypeStruct((NAP,), jnp.int32),
            jax.ShapeDtypeStruct((P,), jnp.int32),
        ],
        scratch_types=[
            pltpu.VMEM((NA,), jnp.int32),
            pltpu.VMEM((E,), jnp.int32),
            pltpu.VMEM((E,), jnp.int32),
            pltpu.VMEM((E,), jnp.int32),
            pltpu.VMEM((E,), jnp.int32),
            pltpu.VMEM((48,), jnp.int32),
            pltpu.VMEM((NAP,), jnp.int32),
            pltpu.VMEM((P,), jnp.int32),
            pltpu.VMEM((NB + 16,), jnp.int32),
            pltpu.VMEM((NA,), jnp.int32),
            pltpu.VMEM((NA,), jnp.int32),
        ],
    )(_sc_meta_body)
    return k(eid_flat)


def kernel(x, expert_weights, expert_indices, scores, W1, W2):
    x2 = x.reshape(SEQ, D)
    x_d = x2[:TRIM]
    x_s = x2[TRIM:]
    w_s = expert_weights.reshape(SEQ, TOPK)[TRIM:]
    eid = expert_indices.reshape(SEQ, TOPK)[TRIM:].astype(jnp.int32)
    sc_d = scores[:TRIM]

    block_expert, invperm, gather_tok = _sc_metadata(eid.reshape(-1))

    W1b = W1.astype(jnp.bfloat16)
    W2b = W2.astype(jnp.bfloat16)
    xg = _sc_row_gather(x_s, gather_tok, 128)
    out_sorted = pl.pallas_call(
        _grouped_mlp_body,
        grid_spec=pltpu.PrefetchScalarGridSpec(
            num_scalar_prefetch=1,
            grid=(NB,),
            in_specs=[
                pl.BlockSpec((BLK, D), lambda b, s: (b, 0)),
                pl.BlockSpec((1, D, F), lambda b, s: (s[b], 0, 0)),
                pl.BlockSpec((1, F, D), lambda b, s: (s[b], 0, 0)),
            ],
            out_specs=pl.BlockSpec((BLK, D), lambda b, s: (b, 0)),
        ),
        out_shape=jax.ShapeDtypeStruct((P, D), jnp.float32),
    )(block_expert, xg, W1b, W2b)

    y_d, buffer = pl.pallas_call(
        _dense_body,
        grid=(E // EG,),
        in_specs=[
            pl.BlockSpec((TRIM, D), lambda e: (0, 0)),
            pl.BlockSpec((EG, D, F), lambda e: (e, 0, 0)),
            pl.BlockSpec((EG, F, D), lambda e: (e, 0, 0)),
            pl.BlockSpec((TRIM, E), lambda e: (0, 0)),
        ],
        out_specs=[
            pl.BlockSpec((TRIM, D), lambda e: (0, 0)),
            pl.BlockSpec((TRIM, EG, D), lambda e: (0, e, 0)),
        ],
        out_shape=[
            jax.ShapeDtypeStruct((TRIM, D), jnp.float32),
            jax.ShapeDtypeStruct((SEQ, E, D), jnp.float32),
        ],
    )(x_d, W1b, W2b, sc_d)

    rows = _sc_row_gather(out_sorted, invperm, 128).reshape(
        NAP // TOPK, TOPK, D)
    y_s, buffer = pl.pallas_call(
        _sparse_out_body,
        grid=(NTB,),
        in_specs=[
            pl.BlockSpec((TB, TOPK, D), lambda i: (i, 0, 0)),
            pl.BlockSpec((TB, TOPK), lambda i: (i, 0)),
            pl.BlockSpec((TB, TOPK), lambda i: (i, 0), memory_space=pltpu.SMEM),
            pl.BlockSpec(memory_space=pltpu.MemorySpace.HBM),
        ],
        out_specs=[
            pl.BlockSpec((TB, D), lambda i: (i, 0)),
            pl.BlockSpec((TB, E, D), lambda i: (i + TRIM // TB, 0, 0)),
        ],
        out_shape=[
            jax.ShapeDtypeStruct((NS, D), jnp.float32),
            jax.ShapeDtypeStruct((SEQ, E, D), jnp.float32),
        ],
        input_output_aliases={3: 1},
    )(rows, w_s, eid, buffer)

    x_out = jnp.concatenate([y_d, y_s], axis=0).reshape(SEQ, 1, D)
    return (x_out, jnp.asarray(TRIM, jnp.int32), buffer.reshape(SEQ, E, D))


# speedup vs baseline: 3.7266x; 1.0340x over previous
import functools

import jax
import jax.numpy as jnp
from jax import lax
from jax.experimental import pallas as pl
from jax.experimental.pallas import tpu as pltpu
from jax.experimental.pallas import tpu_sc as plsc

E = 64
TOPK = 2
D = 768
F = 256
SEQ = 2048
TRIM = 32
NS = SEQ - TRIM
NA = NS * TOPK
BLK = 64
NB = 128
P = NB * BLK
TB = 32
NTB = NS // TB
NAP = 4096


def _grouped_mlp_body(be_ref, xg_ref, w1_ref, w2_ref, out_ref):
    @pl.when(pl.program_id(0) < be_ref[NB])
    def _():
        h = jax.nn.gelu(
            jnp.dot(xg_ref[...].astype(jnp.bfloat16), w1_ref[0],
                    preferred_element_type=jnp.float32)
        )
        out_ref[...] = jnp.dot(h.astype(jnp.bfloat16), w2_ref[0],
                               preferred_element_type=jnp.float32)


def _sparse_out_body(rows_ref, w_ref, eid_ref, _buf_in, y_ref, buf_ref):
    rows = rows_ref[...].astype(jnp.float32)
    w = w_ref[...]
    acc = rows[:, 0, :] * w[:, 0:1] + rows[:, 1, :] * w[:, 1:2]
    y_ref[...] = acc
    buf_ref[...] = jnp.zeros((TB, E, D), jnp.float32)
    for t in range(TB):
        e0 = eid_ref[t, 0]
        e1 = eid_ref[t, 1]
        dup = e0 == e1
        both = rows[t, 0, :] + rows[t, 1, :]
        v0 = jnp.where(dup, both, rows[t, 0, :])
        v1 = jnp.where(dup, both, rows[t, 1, :])
        buf_ref[t, e0, :] = v0
        buf_ref[t, e1, :] = v1


EG = 8


def _dense_body(xd_ref, w1_ref, w2_ref, sc_ref, y_ref, buf_ref):
    eg = pl.program_id(0)
    xd = xd_ref[...].astype(jnp.bfloat16)
    r = jax.lax.broadcasted_iota(jnp.int32, (E, EG), 0)
    c = jax.lax.broadcasted_iota(jnp.int32, (E, EG), 1)
    onehot = (r == EG * eg + c).astype(jnp.float32)
    w8 = jnp.dot(sc_ref[...], onehot, preferred_element_type=jnp.float32)

    outs = []
    acc = jnp.zeros((TRIM, D), jnp.float32)
    for j in range(EG):
        h = jax.nn.gelu(
            jnp.dot(xd, w1_ref[j], preferred_element_type=jnp.float32))
        out = jnp.dot(h.astype(jnp.bfloat16), w2_ref[j],
                      preferred_element_type=jnp.float32)
        outs.append(out)
        acc += out * w8[:, j:j + 1]
    buf_ref[...] = jnp.stack(outs, axis=1)

    @pl.when(eg == 0)
    def _():
        y_ref[...] = jnp.zeros((TRIM, D), jnp.float32)

    y_ref[...] += acc


def _exclusive_cumsum(v):
    return jnp.concatenate([jnp.zeros((1,), v.dtype), jnp.cumsum(v)[:-1]])


_NV = NA // 16
_LANE = None


def _sc_meta_body(eid_hbm, bexp_hbm, inv_hbm, gtok_hbm,
                  ev, hist, base, bstart, counters, tmp, inv_v, gtok_v,
                  bexp_v, occ_v, last_v):
    wid = lax.axis_index("s") * 2 + lax.axis_index("c")

    @pl.when(wid == 0)
    def _():
        iota = lax.iota(jnp.int32, 16)
        zeros16 = jnp.zeros((16,), jnp.int32)
        pltpu.sync_copy(eid_hbm, ev)

        def _z(i, _):
            gtok_v[pl.ds(i * 16, 16)] = (i * 16 + iota) & 1023
            return 0
        lax.fori_loop(0, P // 16, _z, 0)
        for vi in range(4):
            hist[pl.ds(vi * 16, 16)] = zeros16
            counters[pl.ds(vi * 16, 16)] = zeros16
        tmp[pl.ds(0, 16)] = zeros16
        tmp[pl.ds(16, 16)] = zeros16
        tmp[pl.ds(32, 16)] = zeros16
        for vi in range((NAP - NA) // 16):
            inv_v[pl.ds(NA + vi * 16, 16)] = zeros16

        def _hist_step(i, _):
            v = ev[pl.ds(i * 16, 16)]

            def _dup(m, carry):
                occ, cntl = carry
                sp = plsc.load_gather(ev, [jnp.zeros((16,), jnp.int32)
                                           + (i * 16 + m)])
                eq = v == sp
                occ = occ + jnp.where(eq & (iota > m), 1, 0)
                cntl = cntl + jnp.where(eq & (iota < m), 1, 0)
                return occ, cntl
            occ, cntl = lax.fori_loop(0, 16, _dup, (zeros16, zeros16))
            is_last = cntl == 0
            prior = plsc.load_gather(hist, [v])
            plsc.store_scatter(hist, [v], prior + occ + 1, mask=is_last)
            occ_v[pl.ds(i * 16, 16)] = occ
            last_v[pl.ds(i * 16, 16)] = jnp.where(is_last, 1, 0)
            return 0
        lax.fori_loop(0, _NV, _hist_step, 0)

        def _incl_scan(vec):
            cur = vec
            for sh in (1, 2, 4, 8):
                tmp[pl.ds(16, 16)] = cur
                cur = cur + tmp[pl.ds(16 - sh, 16)]
            tmp[pl.ds(32, 16)] = cur
            return cur

        carry_p = jnp.zeros((16,), jnp.int32)
        carry_b = jnp.zeros((16,), jnp.int32)
        full15 = jnp.full((16,), 15, jnp.int32)
        for vi in range(4):
            h = hist[pl.ds(vi * 16, 16)]
            nb = (h + 63) >> 6
            pad = nb * BLK
            incl_p = _incl_scan(pad)
            base[pl.ds(vi * 16, 16)] = incl_p - pad + carry_p
            carry_p = carry_p + plsc.load_gather(tmp, [full15 + 32])
            incl_b = _incl_scan(nb)
            bstart[pl.ds(vi * 16, 16)] = incl_b - nb + carry_b
            carry_b = carry_b + plsc.load_gather(tmp, [full15 + 32])

        for bi in range(NB // 16):
            bvec = bi * 16 + iota

            def _acc(e, a):
                bs_e = plsc.load_gather(bstart, [jnp.zeros((16,), jnp.int32)
                                                 + e])
                return a + jnp.where(bvec >= bs_e, 1, 0)
            bexp_v[pl.ds(bi * 16, 16)] = lax.fori_loop(
                0, E, _acc, jnp.full((16,), -1, jnp.int32))
        bexp_v[pl.ds(NB, 16)] = carry_b

        def _slot_step(i, _):
            v = ev[pl.ds(i * 16, 16)]
            occ = occ_v[pl.ds(i * 16, 16)]
            is_last = last_v[pl.ds(i * 16, 16)] == 1
            prior = plsc.load_gather(counters, [v])
            rank = prior + occ
            plsc.store_scatter(counters, [v], rank + 1, mask=is_last)
            slot = plsc.load_gather(base, [v]) + rank
            jg = i * 16 + iota
            inv_v[pl.ds(i * 16, 16)] = slot
            plsc.store_scatter(gtok_v, [slot], jg >> 1)
            return 0
        lax.fori_loop(0, _NV, _slot_step, 0)

        pltpu.sync_copy(bexp_v, bexp_hbm)
        pltpu.sync_copy(inv_v, inv_hbm)
        pltpu.sync_copy(gtok_v, gtok_hbm)


def _sc_row_gather(table, idx, chunk):
    n_rows, d = idx.shape[0], table.shape[1]
    dt = table.dtype
    nw = 32
    per_w = n_rows // nw
    nch = per_w // chunk
    mesh = plsc.VectorSubcoreMesh(core_axis_name="c", subcore_axis_name="s")

    def body(table_hbm, idx_hbm, out_hbm, idx_v, rows_v, sem):
        wid = lax.axis_index("s") * 2 + lax.axis_index("c")
        base = wid * per_w
        for c in range(nch):
            b = base + c * chunk
            pltpu.sync_copy(idx_hbm.at[pl.ds(b, chunk)], idx_v)
            pltpu.async_copy(table_hbm.at[idx_v], rows_v, sem).wait()
            pltpu.sync_copy(rows_v, out_hbm.at[pl.ds(b, chunk)])

    k = functools.partial(
        pl.kernel, mesh=mesh,
        out_type=jax.ShapeDtypeStruct((n_rows, d), dt),
        scratch_types=[
            pltpu.VMEM((chunk,), jnp.int32),
            pltpu.VMEM((chunk, d), dt),
            pltpu.SemaphoreType.DMA,
        ],
    )(body)
    return k(table, idx)


def _sc_metadata(eid_flat):
    mesh = plsc.VectorSubcoreMesh(core_axis_name="c", subcore_axis_name="s")
    k = functools.partial(
        pl.kernel, mesh=mesh,
        compiler_params=pltpu.CompilerParams(needs_layout_passes=False),
        out_type=[
            jax.ShapeDtypeStruct((NB + 16,), jnp.int32),
            jax.ShapeDtypeStruct((NAP,), jnp.int32),
            jax.ShapeDtypeStruct((P,), jnp.int32),
        ],
        scratch_types=[
            pltpu.VMEM((NA,), jnp.int32),
            pltpu.VMEM((E,), jnp.int32),
            pltpu.VMEM((E,), jnp.int32),
            pltpu.VMEM((E,), jnp.int32),
            pltpu.VMEM((E,), jnp.int32),
            pltpu.VMEM((48,), jnp.int32),
            pltpu.VMEM((NAP,), jnp.int32),
            pltpu.VMEM((P,), jnp.int32),
            pltpu.VMEM((NB + 16,), jnp.int32),
            pltpu.VMEM((NA,), jnp.int32),
            pltpu.VMEM((NA,), jnp.int32),
        ],
    )(_sc_meta_body)
    return k(eid_flat)


def kernel(x, expert_weights, expert_indices, scores, W1, W2):
    x2 = x.reshape(SEQ, D)
    x_d = x2[:TRIM]
    x_s = x2[TRIM:]
    w_s = expert_weights.reshape(SEQ, TOPK)[TRIM:]
    eid = expert_indices.reshape(SEQ, TOPK)[TRIM:].astype(jnp.int32)
    sc_d = scores[:TRIM]

    block_expert, invperm, gather_tok = _sc_metadata(eid.reshape(-1))

    W1b = W1.astype(jnp.bfloat16)
    W2b = W2.astype(jnp.bfloat16)
    xg = _sc_row_gather(x_s, gather_tok, 128)
    out_sorted = pl.pallas_call(
        _grouped_mlp_body,
        grid_spec=pltpu.PrefetchScalarGridSpec(
            num_scalar_prefetch=1,
            grid=(NB,),
            in_specs=[
                pl.BlockSpec((BLK, D), lambda b, s: (b, 0)),
                pl.BlockSpec((1, D, F), lambda b, s: (s[b], 0, 0)),
                pl.BlockSpec((1, F, D), lambda b, s: (s[b], 0, 0)),
            ],
            out_specs=pl.BlockSpec((BLK, D), lambda b, s: (b, 0)),
        ),
        out_shape=jax.ShapeDtypeStruct((P, D), jnp.float32),
    )(block_expert, xg, W1b, W2b)

    y_d, buffer = pl.pallas_call(
        _dense_body,
        grid=(E // EG,),
        in_specs=[
            pl.BlockSpec((TRIM, D), lambda e: (0, 0)),
            pl.BlockSpec((EG, D, F), lambda e: (e, 0, 0)),
            pl.BlockSpec((EG, F, D), lambda e: (e, 0, 0)),
            pl.BlockSpec((TRIM, E), lambda e: (0, 0)),
        ],
        out_specs=[
            pl.BlockSpec((TRIM, D), lambda e: (0, 0)),
            pl.BlockSpec((TRIM, EG, D), lambda e: (0, e, 0)),
        ],
        out_shape=[
            jax.ShapeDtypeStruct((TRIM, D), jnp.float32),
            jax.ShapeDtypeStruct((SEQ, E, D), jnp.float32),
        ],
    )(x_d, W1b, W2b, sc_d)

    rows = _sc_row_gather(out_sorted, invperm, 128).reshape(
        NAP // TOPK, TOPK, D)
    y_s, buffer = pl.pallas_call(
        _sparse_out_body,
        grid=(NTB,),
        in_specs=[
            pl.BlockSpec((TB, TOPK, D), lambda i: (i, 0, 0)),
            pl.BlockSpec((TB, TOPK), lambda i: (i, 0)),
            pl.BlockSpec((TB, TOPK), lambda i: (i, 0), memory_space=pltpu.SMEM),
            pl.BlockSpec(memory_space=pltpu.MemorySpace.HBM),
        ],
        out_specs=[
            pl.BlockSpec((TB, D), lambda i: (i, 0)),
            pl.BlockSpec((TB, E, D), lambda i: (i + TRIM // TB, 0, 0)),
        ],
        out_shape=[
            jax.ShapeDtypeStruct((NS, D), jnp.float32),
            jax.ShapeDtypeStruct((SEQ, E, D), jnp.float32),
        ],
        input_output_aliases={3: 1},
    )(rows, w_s, eid, buffer)

    x_out = jnp.concatenate([y_d, y_s], axis=0).reshape(SEQ, 1, D)
    return (x_out, jnp.asarray(TRIM, jnp.int32), buffer.reshape(SEQ, E, D))


# speedup vs baseline: 3.8156x; 1.0239x over previous
import functools

import jax
import jax.numpy as jnp
from jax import lax
from jax.experimental import pallas as pl
from jax.experimental.pallas import tpu as pltpu
from jax.experimental.pallas import tpu_sc as plsc

E = 64
TOPK = 2
D = 768
F = 256
SEQ = 2048
TRIM = 32
NS = SEQ - TRIM
NA = NS * TOPK
BLK = 64
NB = 128
P = NB * BLK
TB = 32
NTB = NS // TB
NAP = 4096


def _grouped_mlp_body(be_ref, xg_ref, w1_ref, w2_ref, out_ref):
    @pl.when(pl.program_id(0) < be_ref[NB])
    def _():
        h = jax.nn.gelu(
            jnp.dot(xg_ref[...].astype(jnp.bfloat16), w1_ref[0],
                    preferred_element_type=jnp.float32)
        )
        out_ref[...] = jnp.dot(h.astype(jnp.bfloat16), w2_ref[0],
                               preferred_element_type=jnp.float32)


def _sparse_out_body(rows_ref, w_ref, eid_ref, _buf_in, y_ref, buf_ref):
    rows = rows_ref[...].astype(jnp.float32)
    w = w_ref[...]
    acc = rows[:, 0, :] * w[:, 0:1] + rows[:, 1, :] * w[:, 1:2]
    y_ref[...] = acc
    buf_ref[...] = jnp.zeros((TB, E, D), jnp.float32)
    for t in range(TB):
        e0 = eid_ref[t, 0]
        e1 = eid_ref[t, 1]
        dup = e0 == e1
        both = rows[t, 0, :] + rows[t, 1, :]
        v0 = jnp.where(dup, both, rows[t, 0, :])
        v1 = jnp.where(dup, both, rows[t, 1, :])
        buf_ref[t, e0, :] = v0
        buf_ref[t, e1, :] = v1


EG = 8


def _dense_body(xd_ref, w1_ref, w2_ref, sc_ref, y_ref, buf_ref):
    eg = pl.program_id(0)
    xd = xd_ref[...].astype(jnp.bfloat16)
    r = jax.lax.broadcasted_iota(jnp.int32, (E, EG), 0)
    c = jax.lax.broadcasted_iota(jnp.int32, (E, EG), 1)
    onehot = (r == EG * eg + c).astype(jnp.float32)
    w8 = jnp.dot(sc_ref[...], onehot, preferred_element_type=jnp.float32)

    outs = []
    acc = jnp.zeros((TRIM, D), jnp.float32)
    for j in range(EG):
        h = jax.nn.gelu(
            jnp.dot(xd, w1_ref[j], preferred_element_type=jnp.float32))
        out = jnp.dot(h.astype(jnp.bfloat16), w2_ref[j],
                      preferred_element_type=jnp.float32)
        outs.append(out)
        acc += out * w8[:, j:j + 1]
    buf_ref[...] = jnp.stack(outs, axis=1)

    @pl.when(eg == 0)
    def _():
        y_ref[...] = jnp.zeros((TRIM, D), jnp.float32)

    y_ref[...] += acc


_NV = NA // 16


def _sc_meta_body(eid_hbm, bexp_hbm, inv_hbm, gtok_hbm,
                  ev, hist, base, bstart, counters, tmp, inv_v, gtok_v,
                  bexp_v, occ_v, last_v):
    wid = lax.axis_index("s") * 2 + lax.axis_index("c")

    @pl.when(wid == 0)
    def _():
        iota = lax.iota(jnp.int32, 16)
        zeros16 = jnp.zeros((16,), jnp.int32)
        pltpu.sync_copy(eid_hbm, ev)

        def _z(i, _):
            gtok_v[pl.ds(i * 16, 16)] = (i * 16 + iota) & 1023
            return 0
        lax.fori_loop(0, P // 16, _z, 0)
        for vi in range(4):
            hist[pl.ds(vi * 16, 16)] = zeros16
            counters[pl.ds(vi * 16, 16)] = zeros16
        tmp[pl.ds(0, 16)] = zeros16
        tmp[pl.ds(16, 16)] = zeros16
        tmp[pl.ds(32, 16)] = zeros16
        for vi in range((NAP - NA) // 16):
            inv_v[pl.ds(NA + vi * 16, 16)] = zeros16

        def _hist_step(i, _):
            v = ev[pl.ds(i * 16, 16)]

            def _dup(m, carry):
                occ, cntl = carry
                sp = plsc.load_gather(ev, [jnp.zeros((16,), jnp.int32)
                                           + (i * 16 + m)])
                eq = v == sp
                occ = occ + jnp.where(eq & (iota > m), 1, 0)
                cntl = cntl + jnp.where(eq & (iota < m), 1, 0)
                return occ, cntl
            occ, cntl = lax.fori_loop(0, 16, _dup, (zeros16, zeros16))
            is_last = cntl == 0
            prior = plsc.load_gather(hist, [v])
            plsc.store_scatter(hist, [v], prior + occ + 1, mask=is_last)
            occ_v[pl.ds(i * 16, 16)] = occ
            last_v[pl.ds(i * 16, 16)] = jnp.where(is_last, 1, 0)
            return 0
        lax.fori_loop(0, _NV, _hist_step, 0)

        def _incl_scan(vec):
            cur = vec
            for sh in (1, 2, 4, 8):
                tmp[pl.ds(16, 16)] = cur
                cur = cur + tmp[pl.ds(16 - sh, 16)]
            tmp[pl.ds(32, 16)] = cur
            return cur

        carry_p = jnp.zeros((16,), jnp.int32)
        carry_b = jnp.zeros((16,), jnp.int32)
        full15 = jnp.full((16,), 15, jnp.int32)
        for vi in range(4):
            h = hist[pl.ds(vi * 16, 16)]
            nb = (h + 63) >> 6
            pad = nb * BLK
            incl_p = _incl_scan(pad)
            base[pl.ds(vi * 16, 16)] = incl_p - pad + carry_p
            carry_p = carry_p + plsc.load_gather(tmp, [full15 + 32])
            incl_b = _incl_scan(nb)
            bstart[pl.ds(vi * 16, 16)] = incl_b - nb + carry_b
            carry_b = carry_b + plsc.load_gather(tmp, [full15 + 32])

        for bi in range(NB // 16):
            bvec = bi * 16 + iota

            def _acc(e, a):
                bs_e = plsc.load_gather(bstart, [jnp.zeros((16,), jnp.int32)
                                                 + e])
                return a + jnp.where(bvec >= bs_e, 1, 0)
            bexp_v[pl.ds(bi * 16, 16)] = lax.fori_loop(
                0, E, _acc, jnp.full((16,), -1, jnp.int32))
        bexp_v[pl.ds(NB, 16)] = carry_b

        def _slot_step(i, _):
            v = ev[pl.ds(i * 16, 16)]
            occ = occ_v[pl.ds(i * 16, 16)]
            is_last = last_v[pl.ds(i * 16, 16)] == 1
            prior = plsc.load_gather(counters, [v])
            rank = prior + occ
            plsc.store_scatter(counters, [v], rank + 1, mask=is_last)
            slot = plsc.load_gather(base, [v]) + rank
            jg = i * 16 + iota
            inv_v[pl.ds(i * 16, 16)] = slot
            plsc.store_scatter(gtok_v, [slot], jg >> 1)
            return 0
        lax.fori_loop(0, _NV, _slot_step, 0)

        pltpu.sync_copy(bexp_v, bexp_hbm)
        pltpu.sync_copy(inv_v, inv_hbm)
        pltpu.sync_copy(gtok_v, gtok_hbm)


def _sc_row_gather(table, idx, chunk):
    n_rows, d = idx.shape[0], table.shape[1]
    dt = table.dtype
    nw = 32
    per_w = n_rows // nw
    nch = per_w // chunk
    mesh = plsc.VectorSubcoreMesh(core_axis_name="c", subcore_axis_name="s")

    def body(table_hbm, idx_hbm, out_hbm, idx_v, rows_v, sem):
        wid = lax.axis_index("s") * 2 + lax.axis_index("c")
        base = wid * per_w
        for c in range(nch):
            b = base + c * chunk
            pltpu.sync_copy(idx_hbm.at[pl.ds(b, chunk)], idx_v)
            pltpu.async_copy(table_hbm.at[idx_v], rows_v, sem).wait()
            pltpu.sync_copy(rows_v, out_hbm.at[pl.ds(b, chunk)])

    k = functools.partial(
        pl.kernel, mesh=mesh,
        out_type=jax.ShapeDtypeStruct((n_rows, d), dt),
        scratch_types=[
            pltpu.VMEM((chunk,), jnp.int32),
            pltpu.VMEM((chunk, d), dt),
            pltpu.SemaphoreType.DMA,
        ],
    )(body)
    return k(table, idx)


def _sc_metadata(eid_flat):
    mesh = plsc.VectorSubcoreMesh(core_axis_name="c", subcore_axis_name="s")
    k = functools.partial(
        pl.kernel, mesh=mesh,
        compiler_params=pltpu.CompilerParams(needs_layout_passes=False),
        out_type=[
            jax.ShapeDtypeStruct((NB + 16,), jnp.int32),
            jax.ShapeDtypeStruct((NAP,), jnp.int32),
            jax.ShapeDtypeStruct((P,), jnp.int32),
        ],
        scratch_types=[
            pltpu.VMEM((NA,), jnp.int32),
            pltpu.VMEM((E,), jnp.int32),
            pltpu.VMEM((E,), jnp.int32),
            pltpu.VMEM((E,), jnp.int32),
            pltpu.VMEM((E,), jnp.int32),
            pltpu.VMEM((48,), jnp.int32),
            pltpu.VMEM((NAP,), jnp.int32),
            pltpu.VMEM((P,), jnp.int32),
            pltpu.VMEM((NB + 16,), jnp.int32),
            pltpu.VMEM((NA,), jnp.int32),
            pltpu.VMEM((NA,), jnp.int32),
        ],
    )(_sc_meta_body)
    return k(eid_flat)


def kernel(x, expert_weights, expert_indices, scores, W1, W2):
    x2 = x.reshape(SEQ, D)
    x_d = x2[:TRIM]
    x_s = x2[TRIM:]
    w_s = expert_weights.reshape(SEQ, TOPK)[TRIM:]
    eid = expert_indices.reshape(SEQ, TOPK)[TRIM:].astype(jnp.int32)
    sc_d = scores[:TRIM]

    block_expert, invperm, gather_tok = _sc_metadata(eid.reshape(-1))

    W1b = W1.astype(jnp.bfloat16)
    W2b = W2.astype(jnp.bfloat16)
    xg = _sc_row_gather(x_s, gather_tok, 128)
    out_sorted = pl.pallas_call(
        _grouped_mlp_body,
        grid_spec=pltpu.PrefetchScalarGridSpec(
            num_scalar_prefetch=1,
            grid=(NB,),
            in_specs=[
                pl.BlockSpec((BLK, D), lambda b, s: (b, 0)),
                pl.BlockSpec((1, D, F), lambda b, s: (s[b], 0, 0)),
                pl.BlockSpec((1, F, D), lambda b, s: (s[b], 0, 0)),
            ],
            out_specs=pl.BlockSpec((BLK, D), lambda b, s: (b, 0)),
        ),
        out_shape=jax.ShapeDtypeStruct((P, D), jnp.float32),
    )(block_expert, xg, W1b, W2b)

    y_d, buffer = pl.pallas_call(
        _dense_body,
        grid=(E // EG,),
        in_specs=[
            pl.BlockSpec((TRIM, D), lambda e: (0, 0)),
            pl.BlockSpec((EG, D, F), lambda e: (e, 0, 0)),
            pl.BlockSpec((EG, F, D), lambda e: (e, 0, 0)),
            pl.BlockSpec((TRIM, E), lambda e: (0, 0)),
        ],
        out_specs=[
            pl.BlockSpec((TRIM, D), lambda e: (0, 0)),
            pl.BlockSpec((TRIM, EG, D), lambda e: (0, e, 0)),
        ],
        out_shape=[
            jax.ShapeDtypeStruct((TRIM, D), jnp.float32),
            jax.ShapeDtypeStruct((SEQ, E, D), jnp.float32),
        ],
    )(x_d, W1b, W2b, sc_d)

    rows = _sc_row_gather(out_sorted, invperm, 128).reshape(
        NAP // TOPK, TOPK, D)
    y_s, buffer = pl.pallas_call(
        _sparse_out_body,
        grid=(NTB,),
        in_specs=[
            pl.BlockSpec((TB, TOPK, D), lambda i: (i, 0, 0)),
            pl.BlockSpec((TB, TOPK), lambda i: (i, 0)),
            pl.BlockSpec((TB, TOPK), lambda i: (i, 0), memory_space=pltpu.SMEM),
            pl.BlockSpec(memory_space=pltpu.MemorySpace.HBM),
        ],
        out_specs=[
            pl.BlockSpec((TB, D), lambda i: (i, 0)),
            pl.BlockSpec((TB, E, D), lambda i: (i + TRIM // TB, 0, 0)),
        ],
        out_shape=[
            jax.ShapeDtypeStruct((NS, D), jnp.float32),
            jax.ShapeDtypeStruct((SEQ, E, D), jnp.float32),
        ],
        input_output_aliases={3: 1},
    )(rows, w_s, eid, buffer)

    x_out = jnp.concatenate([y_d, y_s], axis=0).reshape(SEQ, 1, D)
    return (x_out, jnp.asarray(TRIM, jnp.int32), buffer.reshape(SEQ, E, D))
